# speedup vs baseline: 1.0002x; 1.0002x over previous
import jax
import jax.numpy as jnp
from jax.experimental import pallas as pl

N_NODES = 10000
N_EDGES = 320000
IN_DIM = 128
HID = 16
HEADS = 8
OUT_DIM = 64
NEG_SLOPE = 0.2


def _gat_layer(x, src, dst, W, att_src, att_dst, heads, out_ch, concat, n_nodes):
    h = x @ W
    h = h.reshape(-1, heads, out_ch)
    x_j = h[src]
    x_i = h[dst]
    alpha = (x_i * att_src).sum(-1) + (x_j * att_dst).sum(-1)
    alpha = jax.nn.leaky_relu(alpha, NEG_SLOPE)
    amax = jax.ops.segment_max(alpha, dst, num_segments=n_nodes)
    amax = jnp.where(jnp.isfinite(amax), amax, 0.0)
    e = jnp.exp(alpha - amax[dst])
    denom = jax.ops.segment_sum(e, dst, num_segments=n_nodes)
    a = e / (denom[dst] + 1e-16)
    msg = x_j * a[..., None]
    out = jax.ops.segment_sum(msg, dst, num_segments=n_nodes)
    if concat:
        out = out.reshape(n_nodes, heads * out_ch)
    else:
        out = out.mean(axis=1)
    return out


def _log_softmax_body(o_ref, out_ref):
    o = o_ref[...]
    m = jnp.max(o, axis=1, keepdims=True)
    l = o - m
    out_ref[...] = l - jnp.log(jnp.sum(jnp.exp(l), axis=1, keepdims=True))


def _log_softmax(o):
    n = o.shape[0]
    blk = 2000
    return pl.pallas_call(
        _log_softmax_body,
        grid=(n // blk,),
        in_specs=[pl.BlockSpec((blk, o.shape[1]), lambda i: (i, 0))],
        out_specs=pl.BlockSpec((blk, o.shape[1]), lambda i: (i, 0)),
        out_shape=jax.ShapeDtypeStruct(o.shape, o.dtype),
    )(o)


def kernel(x, edge_index, W1, att_src1, att_dst1, W2, att_src2, att_dst2):
    src = edge_index[0].astype(jnp.int32)
    dst = edge_index[1].astype(jnp.int32)
    h = _gat_layer(x, src, dst, W1, att_src1, att_dst1, HEADS, HID, True, N_NODES)
    h = jax.nn.elu(h)
    h = _gat_layer(h, src, dst, W2, att_src2, att_dst2, 1, OUT_DIM, False, N_NODES)
    return _log_softmax(h)


# speedup vs baseline: 26.8517x; 26.8458x over previous
import functools

import jax
import jax.numpy as jnp
from jax import lax
from jax.experimental import pallas as pl
from jax.experimental.pallas import tpu as pltpu, tpu_sc as plsc

N_NODES = 10000
N_EDGES = 320000
IN_DIM = 128
HID = 16
HEADS = 8
OUT_DIM = 64
NEG_SLOPE = 0.2

NC = 2
NS = 16
NW = NC * NS
EPW = N_EDGES // NW
CH = 80
NCHUNK = EPW // CH
STRIPE = 632

_MESH = plsc.VectorSubcoreMesh(core_axis_name="c", subcore_axis_name="s",
                               num_cores=NC, num_subcores=NS)
_SC_PARAMS = pltpu.CompilerParams(needs_layout_passes=False,
                                  use_tc_tiling_on_sc=False)


def _block_diag_att(att, pad_to=None):
    _, H, C = att.shape
    a = att.reshape(H, C)
    eye = jnp.eye(H, dtype=att.dtype)
    out = (a[:, :, None] * eye[:, None, :]).reshape(H * C, H)
    if pad_to is not None and pad_to > H:
        out = jnp.concatenate([out, jnp.zeros((H * C, pad_to - H), att.dtype)], axis=1)
    return out


def _tc1_body(x_ref, w_ref, a_ref, h_ref, sd_ref, mx_ref):
    h = jnp.dot(x_ref[...], w_ref[...], preferred_element_type=jnp.float32)
    h_ref[...] = h
    sd = jnp.dot(h, a_ref[...], preferred_element_type=jnp.float32)
    sd_ref[...] = sd
    bmax = jnp.max(sd, axis=0, keepdims=True)

    @pl.when(pl.program_id(0) == 0)
    def _():
        mx_ref[...] = bmax

    @pl.when(pl.program_id(0) != 0)
    def _():
        mx_ref[...] = jnp.maximum(mx_ref[...], bmax)


def _tc1(x, W, A):
    n, k = x.shape
    m = W.shape[1]
    blk = 400
    return pl.pallas_call(
        _tc1_body,
        grid=(n // blk,),
        in_specs=[
            pl.BlockSpec((blk, k), lambda i: (i, 0)),
            pl.BlockSpec((k, m), lambda i: (0, 0)),
            pl.BlockSpec((m, 16), lambda i: (0, 0)),
        ],
        out_specs=[
            pl.BlockSpec((blk, m), lambda i: (i, 0)),
            pl.BlockSpec((blk, 16), lambda i: (i, 0)),
            pl.BlockSpec((1, 16), lambda i: (0, 0)),
        ],
        out_shape=[
            jax.ShapeDtypeStruct((n, m), jnp.float32),
            jax.ShapeDtypeStruct((n, 16), jnp.float32),
            jax.ShapeDtypeStruct((1, 16), jnp.float32),
        ],
    )(x, W, A)


def _tc2_body(p_ref, w_ref, a_ref, h_ref, sd_ref, mx_ref):
    hin = p_ref[0] + p_ref[1]
    hin = jnp.where(hin > 0, hin, jnp.exp(jnp.minimum(hin, 0.0)) - 1.0)
    h = jnp.dot(hin, w_ref[...], preferred_element_type=jnp.float32)
    h_ref[...] = h
    sd = jnp.dot(h, a_ref[...], preferred_element_type=jnp.float32)
    sd_ref[...] = sd
    bmax = jnp.max(sd, axis=0, keepdims=True)

    @pl.when(pl.program_id(0) == 0)
    def _():
        mx_ref[...] = bmax

    @pl.when(pl.program_id(0) != 0)
    def _():
        mx_ref[...] = jnp.maximum(mx_ref[...], bmax)


def _tc2(parts, W, A):
    n = parts.shape[1]
    k = parts.shape[2]
    m = W.shape[1]
    blk = 400
    return pl.pallas_call(
        _tc2_body,
        grid=(n // blk,),
        in_specs=[
            pl.BlockSpec((2, blk, k), lambda i: (0, i, 0)),
            pl.BlockSpec((k, m), lambda i: (0, 0)),
            pl.BlockSpec((m, 16), lambda i: (0, 0)),
        ],
        out_specs=[
            pl.BlockSpec((blk, m), lambda i: (i, 0)),
            pl.BlockSpec((blk, 16), lambda i: (i, 0)),
            pl.BlockSpec((1, 16), lambda i: (0, 0)),
        ],
        out_shape=[
            jax.ShapeDtypeStruct((n, m), jnp.float32),
            jax.ShapeDtypeStruct((n, 16), jnp.float32),
            jax.ShapeDtypeStruct((1, 16), jnp.float32),
        ],
    )(parts, W, A)


def _p1_body(den_ref, sd_ref, bp_ref, p_ref):
    d8 = jnp.sum(den_ref[...], axis=0)
    r = -jnp.log(d8 + 1e-16)
    p_ref[...] = jnp.concatenate([sd_ref[:, :8], r.T], axis=1) - bp_ref[...]


def _p1(den_parts, sd, bp):
    n = sd.shape[0]
    return pl.pallas_call(
        _p1_body,
        out_shape=jax.ShapeDtypeStruct((n, 16), jnp.float32),
    )(den_parts, sd, bp)


def _p2_body(den_ref, sd_ref, bp_ref, p_ref):
    d = jnp.sum(den_ref[...], axis=0)
    r = -jnp.log(d + 1e-16)
    bn = d.shape[0]
    p = jnp.concatenate(
        [sd_ref[:, 0:1], r[:, None], jnp.zeros((bn, 14), jnp.float32)], axis=1)
    p_ref[...] = p - bp_ref[...]


def _p2(den_parts, sd, bp):
    n = sd.shape[0]
    return pl.pallas_call(
        _p2_body,
        out_shape=jax.ShapeDtypeStruct((n, 16), jnp.float32),
    )(den_parts, sd, bp)


@functools.partial(
    pl.kernel,
    out_type=jax.ShapeDtypeStruct((NW, HEADS, N_NODES), jnp.float32),
    mesh=_MESH,
    compiler_params=_SC_PARAMS,
    scratch_types=[
        pltpu.VMEM((EPW,), jnp.int32),
        pltpu.VMEM((EPW,), jnp.int32),
        pltpu.VMEM((N_NODES,), jnp.float32),
        pltpu.VMEM((N_NODES,), jnp.float32),
        pltpu.VMEM((N_NODES,), jnp.float32),
        pltpu.VMEM((16,), jnp.float32),
    ],
)
def _sc_denom1(src_hbm, dst_hbm, sdT_hbm, b_hbm, den_out,
               src_v, dst_v, s_tab, d_tab, den_tab, b_v):
    cid = lax.axis_index("c")
    sid = lax.axis_index("s")
    wid = sid * NC + cid
    base = wid * EPW
    pltpu.sync_copy(src_hbm.at[pl.ds(base, EPW)], src_v)
    pltpu.sync_copy(dst_hbm.at[pl.ds(base, EPW)], dst_v)
    zeros16 = jnp.zeros((16,), jnp.float32)

    @pl.loop(0, HEADS)
    def _head(h):
        pltpu.sync_copy(sdT_hbm.at[h], s_tab)
        pltpu.sync_copy(sdT_hbm.at[h + HEADS], d_tab)

        @pl.loop(0, N_NODES // 16)
        def _z(i):
            den_tab[pl.ds(i * 16, 16)] = zeros16

        pltpu.sync_copy(b_hbm.at[h], b_v)
        bh = b_v[...]

        @pl.loop(0, EPW // 16)
        def _g(g):
            di = dst_v[pl.ds(g * 16, 16)]
            sj = src_v[pl.ds(g * 16, 16)]
            t = plsc.load_gather(s_tab, [di]) + plsc.load_gather(d_tab, [sj])
            al = jnp.where(t >= 0, t, t * NEG_SLOPE)
            e = jnp.exp(al - bh)
            plsc.addupdate_scatter(den_tab, [di], e)

        pltpu.sync_copy(den_tab, den_out.at[wid, h])


@functools.partial(
    pl.kernel,
    out_type=jax.ShapeDtypeStruct((NW, N_NODES), jnp.float32),
    mesh=_MESH,
    compiler_params=_SC_PARAMS,
    scratch_types=[
        pltpu.VMEM((EPW,), jnp.int32),
        pltpu.VMEM((EPW,), jnp.int32),
        pltpu.VMEM((N_NODES,), jnp.float32),
        pltpu.VMEM((N_NODES,), jnp.float32),
        pltpu.VMEM((N_NODES,), jnp.float32),
        pltpu.VMEM((16,), jnp.float32),
    ],
)
def _sc_denom2(src_hbm, dst_hbm, sdT_hbm, b_hbm, den_out,
               src_v, dst_v, s_tab, d_tab, den_tab, b_v):
    cid = lax.axis_index("c")
    sid = lax.axis_index("s")
    wid = sid * NC + cid
    base = wid * EPW
    pltpu.sync_copy(src_hbm.at[pl.ds(base, EPW)], src_v)
    pltpu.sync_copy(dst_hbm.at[pl.ds(base, EPW)], dst_v)
    pltpu.sync_copy(sdT_hbm.at[0], s_tab)
    pltpu.sync_copy(sdT_hbm.at[8], d_tab)
    pltpu.sync_copy(b_hbm.at[0], b_v)
    bh = b_v[...]
    zeros16 = jnp.zeros((16,), jnp.float32)

    @pl.loop(0, N_NODES // 16)
    def _z(i):
        den_tab[pl.ds(i * 16, 16)] = zeros16

    @pl.loop(0, EPW // 16)
    def _g(g):
        di = dst_v[pl.ds(g * 16, 16)]
        sj = src_v[pl.ds(g * 16, 16)]
        t = plsc.load_gather(s_tab, [di]) + plsc.load_gather(d_tab, [sj])
        al = jnp.where(t >= 0, t, t * NEG_SLOPE)
        e = jnp.exp(al - bh)
        plsc.addupdate_scatter(den_tab, [di], e)

    pltpu.sync_copy(den_tab, den_out.at[wid])


@functools.partial(
    pl.kernel,
    out_type=jax.ShapeDtypeStruct((NC, N_NODES, HEADS * HID), jnp.float32),
    mesh=_MESH,
    compiler_params=_SC_PARAMS,
    scratch_types=[
        pltpu.VMEM((EPW,), jnp.int32),
        pltpu.VMEM((EPW,), jnp.int32),
        pltpu.VMEM((CH,), jnp.int32),
        pltpu.VMEM((CH, 16), jnp.float32),
        pltpu.VMEM((CH, 16), jnp.float32),
        pltpu.VMEM((CH, HEADS * HID), jnp.float32),
        pltpu.VMEM((CH, HEADS * HID), jnp.float32),
        pltpu.VMEM_SHARED((N_NODES, HEADS * HID), jnp.float32),
        pltpu.SemaphoreType.DMA,
    ],
)
def _sc_msg1(src_hbm, dst_hbm, p_hbm, q_hbm, h_hbm, z_hbm, out_hbm,
             src_v, dst_v, didx, prow, qrow, hrows, msg, acc, sem):
    cid = lax.axis_index("c")
    sid = lax.axis_index("s")
    wid = sid * NC + cid
    base = wid * EPW
    pltpu.sync_copy(src_hbm.at[pl.ds(base, EPW)], src_v)
    pltpu.sync_copy(dst_hbm.at[pl.ds(base, EPW)], dst_v)
    s0 = jnp.minimum(sid * STRIPE, N_NODES - STRIPE)
    pltpu.sync_copy(z_hbm, acc.at[pl.ds(s0, STRIPE)])
    plsc.subcore_barrier()

    lane = lax.iota(jnp.int32, 16)
    sh_idx = (lane & 7) + 8

    @pl.loop(0, NCHUNK)
    def _chunk(c):
        off = c * CH

        @pl.loop(0, CH // 16)
        def _ci(i):
            didx[pl.ds(i * 16, 16)] = dst_v[pl.ds(off + i * 16, 16)]

        pltpu.async_copy(p_hbm.at[didx], prow, sem).wait()
        pltpu.async_copy(q_hbm.at[src_v.at[pl.ds(off, CH)]], qrow, sem).wait()
        pltpu.async_copy(h_hbm.at[src_v.at[pl.ds(off, CH)]], hrows, sem).wait()

        @pl.loop(0, CH)
        def _e(e):
            t = prow[e, :] + qrow[e, :]
            u = jnp.where(t >= 0, t, t * NEG_SLOPE)
            lu = t.at[sh_idx].get(mode="promise_in_bounds")
            a16 = jnp.exp(u + lu)
            for h in range(HEADS):
                ah = a16.at[lane * 0 + h].get(mode="promise_in_bounds")
                msg[e, pl.ds(h * HID, HID)] = ah * hrows[e, pl.ds(h * HID, HID)]

        pltpu.sync_copy(msg, acc.at[didx], add=True)

    plsc.subcore_barrier()
    pltpu.sync_copy(acc.at[pl.ds(s0, STRIPE)], out_hbm.at[cid, pl.ds(s0, STRIPE)])


@functools.partial(
    pl.kernel,
    out_type=jax.ShapeDtypeStruct((NC, N_NODES, OUT_DIM), jnp.float32),
    mesh=_MESH,
    compiler_params=_SC_PARAMS,
    scratch_types=[
        pltpu.VMEM((EPW,), jnp.int32),
        pltpu.VMEM((EPW,), jnp.int32),
        pltpu.VMEM((CH,), jnp.int32),
        pltpu.VMEM((CH, 16), jnp.float32),
        pltpu.VMEM((CH, 16), jnp.float32),
        pltpu.VMEM((CH, OUT_DIM), jnp.float32),
        pltpu.VMEM((CH, OUT_DIM), jnp.float32),
        pltpu.VMEM_SHARED((N_NODES, OUT_DIM), jnp.float32),
        pltpu.SemaphoreType.DMA,
    ],
)
def _sc_msg2(src_hbm, dst_hbm, p_hbm, q_hbm, h_hbm, z_hbm, out_hbm,
             src_v, dst_v, didx, prow, qrow, hrows, msg, acc, sem):
    cid = lax.axis_index("c")
    sid = lax.axis_index("s")
    wid = sid * NC + cid
    base = wid * EPW
    pltpu.sync_copy(src_hbm.at[pl.ds(base, EPW)], src_v)
    pltpu.sync_copy(dst_hbm.at[pl.ds(base, EPW)], dst_v)
    s0 = jnp.minimum(sid * STRIPE, N_NODES - STRIPE)
    pltpu.sync_copy(z_hbm, acc.at[pl.ds(s0, STRIPE)])
    plsc.subcore_barrier()

    lane = lax.iota(jnp.int32, 16)

    @pl.loop(0, NCHUNK)
    def _chunk(c):
        off = c * CH

        @pl.loop(0, CH // 16)
        def _ci(i):
            didx[pl.ds(i * 16, 16)] = dst_v[pl.ds(off + i * 16, 16)]

        pltpu.async_copy(p_hbm.at[didx], prow, sem).wait()
        pltpu.async_copy(q_hbm.at[src_v.at[pl.ds(off, CH)]], qrow, sem).wait()
        pltpu.async_copy(h_hbm.at[src_v.at[pl.ds(off, CH)]], hrows, sem).wait()

        @pl.loop(0, CH)
        def _e(e):
            t = prow[e, :] + qrow[e, :]
            u = jnp.where(t >= 0, t, t * NEG_SLOPE)
            a0 = u.at[lane * 0].get(mode="promise_in_bounds")
            l0 = t.at[lane * 0 + 1].get(mode="promise_in_bounds")
            a16 = jnp.exp(a0 + l0)
            for q in range(OUT_DIM // 16):
                msg[e, pl.ds(q * 16, 16)] = a16 * hrows[e, pl.ds(q * 16, 16)]

        pltpu.sync_copy(msg, acc.at[didx], add=True)

    plsc.subcore_barrier()
    pltpu.sync_copy(acc.at[pl.ds(s0, STRIPE)], out_hbm.at[cid, pl.ds(s0, STRIPE)])


def _fin_body(p_ref, out_ref):
    o = p_ref[0] + p_ref[1]
    m = jnp.max(o, axis=1, keepdims=True)
    l = o - m
    out_ref[...] = l - jnp.log(jnp.sum(jnp.exp(l), axis=1, keepdims=True))


def _fin(parts):
    n, d = parts.shape[1], parts.shape[2]
    blk = 2000
    return pl.pallas_call(
        _fin_body,
        grid=(n // blk,),
        in_specs=[pl.BlockSpec((2, blk, d), lambda i: (0, i, 0))],
        out_specs=pl.BlockSpec((blk, d), lambda i: (i, 0)),
        out_shape=jax.ShapeDtypeStruct((n, d), jnp.float32),
    )(parts)


def kernel(x, edge_index, W1, att_src1, att_dst1, W2, att_src2, att_dst2):
    src = edge_index[0].astype(jnp.int32)
    dst = edge_index[1].astype(jnp.int32)

    A1 = jnp.concatenate([_block_diag_att(att_src1), _block_diag_att(att_dst1)], axis=1)
    h1, sd1, mx1 = _tc1(x, W1, A1)
    b1 = jax.nn.leaky_relu(mx1[0, :HEADS] + mx1[0, HEADS:], NEG_SLOPE)
    b_rows1 = jnp.broadcast_to(b1[:, None], (HEADS, 16))
    den1 = _sc_denom1(src, dst, sd1.T, b_rows1)
    bp1 = jnp.concatenate([jnp.zeros((8,), jnp.float32), b1])[None, :]
    P1 = _p1(den1, sd1, bp1)
    Q1 = jnp.concatenate([sd1[:, HEADS:], jnp.zeros((N_NODES, 8), jnp.float32)], axis=1)
    z1 = jnp.zeros((STRIPE, HEADS * HID), jnp.float32)
    out1 = _sc_msg1(src, dst, P1, Q1, h1, z1)

    A2 = jnp.concatenate([_block_diag_att(att_src2, pad_to=8),
                          _block_diag_att(att_dst2, pad_to=8)], axis=1)
    h2, sd2, mx2 = _tc2(out1, W2, A2)
    b2 = jax.nn.leaky_relu(mx2[0, 0] + mx2[0, 8], NEG_SLOPE)
    b_rows2 = jnp.broadcast_to(b2[None, None], (1, 16))
    den2 = _sc_denom2(src, dst, sd2.T, b_rows2)
    bp2 = jnp.zeros((1, 16), jnp.float32).at[0, 1].set(b2)
    P2 = _p2(den2, sd2, bp2)
    Q2 = jnp.concatenate([sd2[:, 8:9], jnp.zeros((N_NODES, 15), jnp.float32)], axis=1)
    z2 = jnp.zeros((STRIPE, OUT_DIM), jnp.float32)
    out2 = _sc_msg2(src, dst, P2, Q2, h2, z2)

    return _fin(out2)


# speedup vs baseline: 37.2419x; 1.3869x over previous
import functools

import jax
import jax.numpy as jnp
from jax import lax
from jax.experimental import pallas as pl
from jax.experimental.pallas import tpu as pltpu, tpu_sc as plsc

N_NODES = 10000
N_EDGES = 320000
IN_DIM = 128
HID = 16
HEADS = 8
OUT_DIM = 64
NEG_SLOPE = 0.2

NC = 2
NS = 16
NW = NC * NS
EPW = N_EDGES // NW
CH = 200
NCHUNK = EPW // CH
STRIPE = 632

_MESH = plsc.VectorSubcoreMesh(core_axis_name="c", subcore_axis_name="s",
                               num_cores=NC, num_subcores=NS)
_SC_PARAMS = pltpu.CompilerParams(needs_layout_passes=False,
                                  use_tc_tiling_on_sc=False)


def _block_diag_att(att, pad_to=None):
    _, H, C = att.shape
    a = att.reshape(H, C)
    eye = jnp.eye(H, dtype=att.dtype)
    out = (a[:, :, None] * eye[:, None, :]).reshape(H * C, H)
    if pad_to is not None and pad_to > H:
        out = jnp.concatenate([out, jnp.zeros((H * C, pad_to - H), att.dtype)], axis=1)
    return out


def _tc1_body(x_ref, w_ref, a_ref, h_ref, sd_ref, mx_ref):
    h = jnp.dot(x_ref[...], w_ref[...], preferred_element_type=jnp.float32)
    h_ref[...] = h
    sd = jnp.dot(h, a_ref[...], preferred_element_type=jnp.float32)
    sd_ref[...] = sd
    bmax = jnp.max(sd, axis=0, keepdims=True)

    @pl.when(pl.program_id(0) == 0)
    def _():
        mx_ref[...] = bmax

    @pl.when(pl.program_id(0) != 0)
    def _():
        mx_ref[...] = jnp.maximum(mx_ref[...], bmax)


def _tc1(x, W, A):
    n, k = x.shape
    m = W.shape[1]
    blk = 400
    return pl.pallas_call(
        _tc1_body,
        grid=(n // blk,),
        in_specs=[
            pl.BlockSpec((blk, k), lambda i: (i, 0)),
            pl.BlockSpec((k, m), lambda i: (0, 0)),
            pl.BlockSpec((m, 16), lambda i: (0, 0)),
        ],
        out_specs=[
            pl.BlockSpec((blk, m), lambda i: (i, 0)),
            pl.BlockSpec((blk, 16), lambda i: (i, 0)),
            pl.BlockSpec((1, 16), lambda i: (0, 0)),
        ],
        out_shape=[
            jax.ShapeDtypeStruct((n, m), jnp.float32),
            jax.ShapeDtypeStruct((n, 16), jnp.float32),
            jax.ShapeDtypeStruct((1, 16), jnp.float32),
        ],
    )(x, W, A)


def _tc2_body(p_ref, w_ref, a_ref, h_ref, sd_ref, mx_ref):
    hin = p_ref[0] + p_ref[1]
    hin = jnp.where(hin > 0, hin, jnp.exp(jnp.minimum(hin, 0.0)) - 1.0)
    h = jnp.dot(hin, w_ref[...], preferred_element_type=jnp.float32)
    h_ref[...] = h
    sd = jnp.dot(h, a_ref[...], preferred_element_type=jnp.float32)
    sd_ref[...] = sd
    bmax = jnp.max(sd, axis=0, keepdims=True)

    @pl.when(pl.program_id(0) == 0)
    def _():
        mx_ref[...] = bmax

    @pl.when(pl.program_id(0) != 0)
    def _():
        mx_ref[...] = jnp.maximum(mx_ref[...], bmax)


def _tc2(parts, W, A):
    n = parts.shape[1]
    k = parts.shape[2]
    m = W.shape[1]
    blk = 400
    return pl.pallas_call(
        _tc2_body,
        grid=(n // blk,),
        in_specs=[
            pl.BlockSpec((2, blk, k), lambda i: (0, i, 0)),
            pl.BlockSpec((k, m), lambda i: (0, 0)),
            pl.BlockSpec((m, 16), lambda i: (0, 0)),
        ],
        out_specs=[
            pl.BlockSpec((blk, m), lambda i: (i, 0)),
            pl.BlockSpec((blk, 16), lambda i: (i, 0)),
            pl.BlockSpec((1, 16), lambda i: (0, 0)),
        ],
        out_shape=[
            jax.ShapeDtypeStruct((n, m), jnp.float32),
            jax.ShapeDtypeStruct((n, 16), jnp.float32),
            jax.ShapeDtypeStruct((1, 16), jnp.float32),
        ],
    )(parts, W, A)


def _p1_body(den_ref, sd_ref, bp_ref, p_ref):
    d8 = jnp.sum(den_ref[...], axis=0)
    r = -jnp.log(d8 + 1e-16)
    p_ref[...] = jnp.concatenate([sd_ref[:, :8], r.T], axis=1) - bp_ref[...]


def _p1(den_parts, sd, bp):
    n = sd.shape[0]
    return pl.pallas_call(
        _p1_body,
        out_shape=jax.ShapeDtypeStruct((n, 16), jnp.float32),
    )(den_parts, sd, bp)


def _p2_body(den_ref, sd_ref, bp_ref, p_ref):
    d = jnp.sum(den_ref[...], axis=0)
    r = -jnp.log(d + 1e-16)
    bn = d.shape[0]
    p = jnp.concatenate(
        [sd_ref[:, 0:1], r[:, None], jnp.zeros((bn, 14), jnp.float32)], axis=1)
    p_ref[...] = p - bp_ref[...]


def _p2(den_parts, sd, bp):
    n = sd.shape[0]
    return pl.pallas_call(
        _p2_body,
        out_shape=jax.ShapeDtypeStruct((n, 16), jnp.float32),
    )(den_parts, sd, bp)


@functools.partial(
    pl.kernel,
    out_type=jax.ShapeDtypeStruct((NW, HEADS, N_NODES), jnp.float32),
    mesh=_MESH,
    compiler_params=_SC_PARAMS,
    scratch_types=[
        pltpu.VMEM((EPW,), jnp.int32),
        pltpu.VMEM((EPW,), jnp.int32),
        pltpu.VMEM((N_NODES,), jnp.float32),
        pltpu.VMEM((N_NODES,), jnp.float32),
        pltpu.VMEM((N_NODES,), jnp.float32),
        pltpu.VMEM((16,), jnp.float32),
    ],
)
def _sc_denom1(src_hbm, dst_hbm, sdT_hbm, b_hbm, den_out,
               src_v, dst_v, s_tab, d_tab, den_tab, b_v):
    cid = lax.axis_index("c")
    sid = lax.axis_index("s")
    wid = sid * NC + cid
    base = wid * EPW
    pltpu.sync_copy(src_hbm.at[pl.ds(base, EPW)], src_v)
    pltpu.sync_copy(dst_hbm.at[pl.ds(base, EPW)], dst_v)
    zeros16 = jnp.zeros((16,), jnp.float32)

    @pl.loop(0, HEADS)
    def _head(h):
        pltpu.sync_copy(sdT_hbm.at[h], s_tab)
        pltpu.sync_copy(sdT_hbm.at[h + HEADS], d_tab)

        @pl.loop(0, N_NODES // 16)
        def _z(i):
            den_tab[pl.ds(i * 16, 16)] = zeros16

        pltpu.sync_copy(b_hbm.at[h], b_v)
        bh = b_v[...]

        @pl.loop(0, EPW // 16)
        def _g(g):
            di = dst_v[pl.ds(g * 16, 16)]
            sj = src_v[pl.ds(g * 16, 16)]
            t = plsc.load_gather(s_tab, [di]) + plsc.load_gather(d_tab, [sj])
            al = jnp.where(t >= 0, t, t * NEG_SLOPE)
            e = jnp.exp(al - bh)
            plsc.addupdate_scatter(den_tab, [di], e)

        pltpu.sync_copy(den_tab, den_out.at[wid, h])


@functools.partial(
    pl.kernel,
    out_type=jax.ShapeDtypeStruct((NW, N_NODES), jnp.float32),
    mesh=_MESH,
    compiler_params=_SC_PARAMS,
    scratch_types=[
        pltpu.VMEM((EPW,), jnp.int32),
        pltpu.VMEM((EPW,), jnp.int32),
        pltpu.VMEM((N_NODES,), jnp.float32),
        pltpu.VMEM((N_NODES,), jnp.float32),
        pltpu.VMEM((N_NODES,), jnp.float32),
        pltpu.VMEM((16,), jnp.float32),
    ],
)
def _sc_denom2(src_hbm, dst_hbm, sdT_hbm, b_hbm, den_out,
               src_v, dst_v, s_tab, d_tab, den_tab, b_v):
    cid = lax.axis_index("c")
    sid = lax.axis_index("s")
    wid = sid * NC + cid
    base = wid * EPW
    pltpu.sync_copy(src_hbm.at[pl.ds(base, EPW)], src_v)
    pltpu.sync_copy(dst_hbm.at[pl.ds(base, EPW)], dst_v)
    pltpu.sync_copy(sdT_hbm.at[0], s_tab)
    pltpu.sync_copy(sdT_hbm.at[8], d_tab)
    pltpu.sync_copy(b_hbm.at[0], b_v)
    bh = b_v[...]
    zeros16 = jnp.zeros((16,), jnp.float32)

    @pl.loop(0, N_NODES // 16)
    def _z(i):
        den_tab[pl.ds(i * 16, 16)] = zeros16

    @pl.loop(0, EPW // 16)
    def _g(g):
        di = dst_v[pl.ds(g * 16, 16)]
        sj = src_v[pl.ds(g * 16, 16)]
        t = plsc.load_gather(s_tab, [di]) + plsc.load_gather(d_tab, [sj])
        al = jnp.where(t >= 0, t, t * NEG_SLOPE)
        e = jnp.exp(al - bh)
        plsc.addupdate_scatter(den_tab, [di], e)

    pltpu.sync_copy(den_tab, den_out.at[wid])


SUB = 40
NSUB = CH // SUB


def _make_sc_msg(fdim, nheads):
    def body(src_hbm, dst_hbm, p_hbm, q_hbm, h_hbm, z_hbm, out_hbm,
             sch, dch, didx0, didx1, didx2, didx3, didx4, prow, qrow, hrows,
             msg0, msg1, acc, esem, gsem, ssem0, ssem1):
        cid = lax.axis_index("c")
        sid = lax.axis_index("s")
        wid = sid * NC + cid
        base = wid * EPW
        s0 = jnp.minimum(sid * STRIPE, N_NODES - STRIPE)
        pltpu.sync_copy(z_hbm, acc.at[pl.ds(s0, STRIPE)])
        plsc.subcore_barrier()

        lane = lax.iota(jnp.int32, 16)
        sh_idx = (lane & 7) + 8

        def edge_descs(c):
            return (pltpu.make_async_copy(src_hbm.at[pl.ds(base + c * CH, CH)], sch, esem),
                    pltpu.make_async_copy(dst_hbm.at[pl.ds(base + c * CH, CH)], dch, esem))

        for d in edge_descs(0):
            d.start()

        @pl.loop(0, NCHUNK)
        def _chunk(c):
            for d in edge_descs(c):
                d.wait()
            didxs = [didx0, didx1, didx2, didx3, didx4]
            offs = [0, 16, SUB - 16]
            for r in range(NSUB):
                for k in offs:
                    didxs[r][pl.ds(k, 16)] = dch[pl.ds(r * SUB + k, 16)]
            gds = []
            for r in range(NSUB):
                rs = pl.ds(r * SUB, SUB)
                for tab, dstbuf in ((p_hbm.at[didxs[r]], prow.at[rs]),
                                    (q_hbm.at[sch.at[rs]], qrow.at[rs]),
                                    (h_hbm.at[sch.at[rs]], hrows.at[rs])):
                    d = pltpu.make_async_copy(tab, dstbuf, gsem)
                    d.start()
                    gds.append(d)
            for d in gds:
                d.wait()

            @pl.when(c + 1 < NCHUNK)
            def _():
                for d in edge_descs(c + 1):
                    d.start()

            sdescs = []
            for r in range(NSUB):
                mb, ssem = (msg0, ssem0) if r % 2 == 0 else (msg1, ssem1)
                if r >= 2:
                    sdescs[r - 2].wait()

                @pl.loop(0, SUB, unroll=4)
                def _e(e, _r=r, _mb=mb):
                    ge = _r * SUB + e
                    t = prow[ge, :] + qrow[ge, :]
                    u = jnp.where(t >= 0, t, t * NEG_SLOPE)
                    if nheads == 8:
                        lu = t.at[sh_idx].get(mode="promise_in_bounds")
                        a16 = jnp.exp(u + lu)
                        for h in range(8):
                            ah = a16.at[lane * 0 + h].get(mode="promise_in_bounds")
                            _mb[e, pl.ds(h * 16, 16)] = ah * hrows[ge, pl.ds(h * 16, 16)]
                    else:
                        a0 = u.at[lane * 0].get(mode="promise_in_bounds")
                        l0 = t.at[lane * 0 + 1].get(mode="promise_in_bounds")
                        a16 = jnp.exp(a0 + l0)
                        for qd in range(fdim // 16):
                            _mb[e, pl.ds(qd * 16, 16)] = a16 * hrows[ge, pl.ds(qd * 16, 16)]

                sd = pltpu.make_async_copy(mb, acc.at[didxs[r]], ssem)
                sd.start(add=True)
                sdescs.append(sd)
            sdescs[NSUB - 2].wait()
            sdescs[NSUB - 1].wait()

        plsc.subcore_barrier()
        pltpu.sync_copy(acc.at[pl.ds(s0, STRIPE)], out_hbm.at[cid, pl.ds(s0, STRIPE)])

    return pl.kernel(
        body,
        out_type=jax.ShapeDtypeStruct((NC, N_NODES, fdim), jnp.float32),
        mesh=_MESH,
        compiler_params=_SC_PARAMS,
        scratch_types=[
            pltpu.VMEM((CH,), jnp.int32),
            pltpu.VMEM((CH,), jnp.int32),
            pltpu.VMEM((SUB,), jnp.int32),
            pltpu.VMEM((SUB,), jnp.int32),
            pltpu.VMEM((SUB,), jnp.int32),
            pltpu.VMEM((SUB,), jnp.int32),
            pltpu.VMEM((SUB,), jnp.int32),
            pltpu.VMEM((CH, 16), jnp.float32),
            pltpu.VMEM((CH, 16), jnp.float32),
            pltpu.VMEM((CH, fdim), jnp.float32),
            pltpu.VMEM((SUB, fdim), jnp.float32),
            pltpu.VMEM((SUB, fdim), jnp.float32),
            pltpu.VMEM_SHARED((N_NODES, fdim), jnp.float32),
            pltpu.SemaphoreType.DMA,
            pltpu.SemaphoreType.DMA,
            pltpu.SemaphoreType.DMA,
            pltpu.SemaphoreType.DMA,
        ],
    )


_sc_msg1 = _make_sc_msg(HEADS * HID, HEADS)
_sc_msg2 = _make_sc_msg(OUT_DIM, 1)


def _fin_body(p_ref, out_ref):
    o = p_ref[0] + p_ref[1]
    m = jnp.max(o, axis=1, keepdims=True)
    l = o - m
    out_ref[...] = l - jnp.log(jnp.sum(jnp.exp(l), axis=1, keepdims=True))


def _fin(parts):
    n, d = parts.shape[1], parts.shape[2]
    blk = 2000
    return pl.pallas_call(
        _fin_body,
        grid=(n // blk,),
        in_specs=[pl.BlockSpec((2, blk, d), lambda i: (0, i, 0))],
        out_specs=pl.BlockSpec((blk, d), lambda i: (i, 0)),
        out_shape=jax.ShapeDtypeStruct((n, d), jnp.float32),
    )(parts)


def kernel(x, edge_index, W1, att_src1, att_dst1, W2, att_src2, att_dst2):
    src = edge_index[0].astype(jnp.int32)
    dst = edge_index[1].astype(jnp.int32)

    A1 = jnp.concatenate([_block_diag_att(att_src1), _block_diag_att(att_dst1)], axis=1)
    h1, sd1, mx1 = _tc1(x, W1, A1)
    b1 = jax.nn.leaky_relu(mx1[0, :HEADS] + mx1[0, HEADS:], NEG_SLOPE)
    b_rows1 = jnp.broadcast_to(b1[:, None], (HEADS, 16))
    den1 = _sc_denom1(src, dst, sd1.T, b_rows1)
    bp1 = jnp.concatenate([jnp.zeros((8,), jnp.float32), b1])[None, :]
    P1 = _p1(den1, sd1, bp1)
    Q1 = jnp.concatenate([sd1[:, HEADS:], jnp.zeros((N_NODES, 8), jnp.float32)], axis=1)
    z1 = jnp.zeros((STRIPE, HEADS * HID), jnp.float32)
    out1 = _sc_msg1(src, dst, P1, Q1, h1, z1)

    A2 = jnp.concatenate([_block_diag_att(att_src2, pad_to=8),
                          _block_diag_att(att_dst2, pad_to=8)], axis=1)
    h2, sd2, mx2 = _tc2(out1, W2, A2)
    b2 = jax.nn.leaky_relu(mx2[0, 0] + mx2[0, 8], NEG_SLOPE)
    b_rows2 = jnp.broadcast_to(b2[None, None], (1, 16))
    den2 = _sc_denom2(src, dst, sd2.T, b_rows2)
    bp2 = jnp.zeros((1, 16), jnp.float32).at[0, 1].set(b2)
    P2 = _p2(den2, sd2, bp2)
    Q2 = jnp.concatenate([sd2[:, 8:9], jnp.zeros((N_NODES, 15), jnp.float32)], axis=1)
    z2 = jnp.zeros((STRIPE, OUT_DIM), jnp.float32)
    out2 = _sc_msg2(src, dst, P2, Q2, h2, z2)

    return _fin(out2)


# speedup vs baseline: 39.6379x; 1.0643x over previous
import functools

import jax
import jax.numpy as jnp
from jax import lax
from jax.experimental import pallas as pl
from jax.experimental.pallas import tpu as pltpu, tpu_sc as plsc

N_NODES = 10000
N_EDGES = 320000
IN_DIM = 128
HID = 16
HEADS = 8
OUT_DIM = 64
NEG_SLOPE = 0.2

NC = 2
NS = 16
NW = NC * NS
EPW = N_EDGES // NW
CH = 200
NCHUNK = EPW // CH
STRIPE = 632

_MESH = plsc.VectorSubcoreMesh(core_axis_name="c", subcore_axis_name="s",
                               num_cores=NC, num_subcores=NS)
_SC_PARAMS = pltpu.CompilerParams(needs_layout_passes=False,
                                  use_tc_tiling_on_sc=False)


def _block_diag_att(att, pad_to=None):
    _, H, C = att.shape
    a = att.reshape(H, C)
    eye = jnp.eye(H, dtype=att.dtype)
    out = (a[:, :, None] * eye[:, None, :]).reshape(H * C, H)
    if pad_to is not None and pad_to > H:
        out = jnp.concatenate([out, jnp.zeros((H * C, pad_to - H), att.dtype)], axis=1)
    return out


def _tc1_body(x_ref, w_ref, a_ref, h_ref, sd_ref, mx_ref):
    h = jnp.dot(x_ref[...], w_ref[...], preferred_element_type=jnp.float32)
    h_ref[...] = h
    sd = jnp.dot(h, a_ref[...], preferred_element_type=jnp.float32)
    sd_ref[...] = sd
    bmax = jnp.max(sd, axis=0, keepdims=True)

    @pl.when(pl.program_id(0) == 0)
    def _():
        mx_ref[...] = bmax

    @pl.when(pl.program_id(0) != 0)
    def _():
        mx_ref[...] = jnp.maximum(mx_ref[...], bmax)


def _tc1(x, W, A):
    n, k = x.shape
    m = W.shape[1]
    blk = 400
    return pl.pallas_call(
        _tc1_body,
        grid=(n // blk,),
        in_specs=[
            pl.BlockSpec((blk, k), lambda i: (i, 0)),
            pl.BlockSpec((k, m), lambda i: (0, 0)),
            pl.BlockSpec((m, 16), lambda i: (0, 0)),
        ],
        out_specs=[
            pl.BlockSpec((blk, m), lambda i: (i, 0)),
            pl.BlockSpec((blk, 16), lambda i: (i, 0)),
            pl.BlockSpec((1, 16), lambda i: (0, 0)),
        ],
        out_shape=[
            jax.ShapeDtypeStruct((n, m), jnp.float32),
            jax.ShapeDtypeStruct((n, 16), jnp.float32),
            jax.ShapeDtypeStruct((1, 16), jnp.float32),
        ],
    )(x, W, A)


def _tc2_body(p_ref, w_ref, a_ref, h_ref, sd_ref, mx_ref):
    hin = p_ref[0] + p_ref[1]
    hin = jnp.where(hin > 0, hin, jnp.exp(jnp.minimum(hin, 0.0)) - 1.0)
    h = jnp.dot(hin, w_ref[...], preferred_element_type=jnp.float32)
    h_ref[...] = h
    sd = jnp.dot(h, a_ref[...], preferred_element_type=jnp.float32)
    sd_ref[...] = sd
    bmax = jnp.max(sd, axis=0, keepdims=True)

    @pl.when(pl.program_id(0) == 0)
    def _():
        mx_ref[...] = bmax

    @pl.when(pl.program_id(0) != 0)
    def _():
        mx_ref[...] = jnp.maximum(mx_ref[...], bmax)


def _tc2(parts, W, A):
    n = parts.shape[1]
    k = parts.shape[2]
    m = W.shape[1]
    blk = 400
    return pl.pallas_call(
        _tc2_body,
        grid=(n // blk,),
        in_specs=[
            pl.BlockSpec((2, blk, k), lambda i: (0, i, 0)),
            pl.BlockSpec((k, m), lambda i: (0, 0)),
            pl.BlockSpec((m, 16), lambda i: (0, 0)),
        ],
        out_specs=[
            pl.BlockSpec((blk, m), lambda i: (i, 0)),
            pl.BlockSpec((blk, 16), lambda i: (i, 0)),
            pl.BlockSpec((1, 16), lambda i: (0, 0)),
        ],
        out_shape=[
            jax.ShapeDtypeStruct((n, m), jnp.float32),
            jax.ShapeDtypeStruct((n, 16), jnp.float32),
            jax.ShapeDtypeStruct((1, 16), jnp.float32),
        ],
    )(parts, W, A)


def _p1_body(den_ref, sd_ref, bp_ref, p_ref):
    d8 = jnp.sum(den_ref[...], axis=0)
    r = -jnp.log(d8 + 1e-16)
    p_ref[...] = jnp.concatenate([sd_ref[:, :8], r.T], axis=1) - bp_ref[...]


def _p1(den_parts, sd, bp):
    n = sd.shape[0]
    return pl.pallas_call(
        _p1_body,
        out_shape=jax.ShapeDtypeStruct((n, 16), jnp.float32),
    )(den_parts, sd, bp)


def _p2_body(den_ref, sd_ref, bp_ref, p_ref):
    d = jnp.sum(den_ref[...], axis=0)
    r = -jnp.log(d + 1e-16)
    bn = d.shape[0]
    p = jnp.concatenate(
        [sd_ref[:, 0:1], r[:, None], jnp.zeros((bn, 14), jnp.float32)], axis=1)
    p_ref[...] = p - bp_ref[...]


def _p2(den_parts, sd, bp):
    n = sd.shape[0]
    return pl.pallas_call(
        _p2_body,
        out_shape=jax.ShapeDtypeStruct((n, 16), jnp.float32),
    )(den_parts, sd, bp)


@functools.partial(
    pl.kernel,
    out_type=jax.ShapeDtypeStruct((NW, HEADS, N_NODES), jnp.float32),
    mesh=_MESH,
    compiler_params=_SC_PARAMS,
    scratch_types=[
        pltpu.VMEM((EPW,), jnp.int32),
        pltpu.VMEM((EPW,), jnp.int32),
        pltpu.VMEM((N_NODES,), jnp.float32),
        pltpu.VMEM((N_NODES,), jnp.float32),
        pltpu.VMEM((N_NODES,), jnp.float32),
        pltpu.VMEM((16,), jnp.float32),
    ],
)
def _sc_denom1(src_hbm, dst_hbm, sdT_hbm, b_hbm, den_out,
               src_v, dst_v, s_tab, d_tab, den_tab, b_v):
    cid = lax.axis_index("c")
    sid = lax.axis_index("s")
    wid = sid * NC + cid
    base = wid * EPW
    pltpu.sync_copy(src_hbm.at[pl.ds(base, EPW)], src_v)
    pltpu.sync_copy(dst_hbm.at[pl.ds(base, EPW)], dst_v)
    zeros16 = jnp.zeros((16,), jnp.float32)

    @pl.loop(0, HEADS)
    def _head(h):
        pltpu.sync_copy(sdT_hbm.at[h], s_tab)
        pltpu.sync_copy(sdT_hbm.at[h + HEADS], d_tab)

        @pl.loop(0, N_NODES // 16)
        def _z(i):
            den_tab[pl.ds(i * 16, 16)] = zeros16

        pltpu.sync_copy(b_hbm.at[h], b_v)
        bh = b_v[...]

        @pl.loop(0, EPW // 16)
        def _g(g):
            di = dst_v[pl.ds(g * 16, 16)]
            sj = src_v[pl.ds(g * 16, 16)]
            t = plsc.load_gather(s_tab, [di]) + plsc.load_gather(d_tab, [sj])
            al = jnp.where(t >= 0, t, t * NEG_SLOPE)
            e = jnp.exp(al - bh)
            plsc.addupdate_scatter(den_tab, [di], e)

        pltpu.sync_copy(den_tab, den_out.at[wid, h])


@functools.partial(
    pl.kernel,
    out_type=jax.ShapeDtypeStruct((NW, N_NODES), jnp.float32),
    mesh=_MESH,
    compiler_params=_SC_PARAMS,
    scratch_types=[
        pltpu.VMEM((EPW,), jnp.int32),
        pltpu.VMEM((EPW,), jnp.int32),
        pltpu.VMEM((N_NODES,), jnp.float32),
        pltpu.VMEM((N_NODES,), jnp.float32),
        pltpu.VMEM((N_NODES,), jnp.float32),
        pltpu.VMEM((16,), jnp.float32),
    ],
)
def _sc_denom2(src_hbm, dst_hbm, sdT_hbm, b_hbm, den_out,
               src_v, dst_v, s_tab, d_tab, den_tab, b_v):
    cid = lax.axis_index("c")
    sid = lax.axis_index("s")
    wid = sid * NC + cid
    base = wid * EPW
    pltpu.sync_copy(src_hbm.at[pl.ds(base, EPW)], src_v)
    pltpu.sync_copy(dst_hbm.at[pl.ds(base, EPW)], dst_v)
    pltpu.sync_copy(sdT_hbm.at[0], s_tab)
    pltpu.sync_copy(sdT_hbm.at[8], d_tab)
    pltpu.sync_copy(b_hbm.at[0], b_v)
    bh = b_v[...]
    zeros16 = jnp.zeros((16,), jnp.float32)

    @pl.loop(0, N_NODES // 16)
    def _z(i):
        den_tab[pl.ds(i * 16, 16)] = zeros16

    @pl.loop(0, EPW // 16)
    def _g(g):
        di = dst_v[pl.ds(g * 16, 16)]
        sj = src_v[pl.ds(g * 16, 16)]
        t = plsc.load_gather(s_tab, [di]) + plsc.load_gather(d_tab, [sj])
        al = jnp.where(t >= 0, t, t * NEG_SLOPE)
        e = jnp.exp(al - bh)
        plsc.addupdate_scatter(den_tab, [di], e)

    pltpu.sync_copy(den_tab, den_out.at[wid])


def _make_sc_msg(fdim, nheads, ch, sub):
    nsub = ch // sub
    nchunk = EPW // ch
    assert nsub == 5 and ch % sub == 0 and EPW % ch == 0
    offs = sorted(set(list(range(0, sub - 15, 16)) + [sub - 16]))

    def body(src_hbm, dst_hbm, p_hbm, q_hbm, h_hbm, z_hbm, out_hbm,
             sch, dch, didx0, didx1, didx2, didx3, didx4, prow, qrow, hrows,
             msg0, msg1, acc, esem, g0, g1, g2, g3, g4, ssem0, ssem1):
        cid = lax.axis_index("c")
        sid = lax.axis_index("s")
        wid = sid * NC + cid
        base = wid * EPW
        s0 = jnp.minimum(sid * STRIPE, N_NODES - STRIPE)
        pltpu.sync_copy(z_hbm, acc.at[pl.ds(s0, STRIPE)])
        plsc.subcore_barrier()

        didxs = [didx0, didx1, didx2, didx3, didx4]
        gsems = [g0, g1, g2, g3, g4]
        lane = lax.iota(jnp.int32, 16)
        sh_idx = (lane & 7) + 8

        def edge_descs(c):
            return (pltpu.make_async_copy(src_hbm.at[pl.ds(base + c * ch, ch)], sch, esem),
                    pltpu.make_async_copy(dst_hbm.at[pl.ds(base + c * ch, ch)], dch, esem))

        def gather_descs(r):
            rs = pl.ds(r * sub, sub)
            return (pltpu.make_async_copy(p_hbm.at[didxs[r]], prow.at[rs], gsems[r]),
                    pltpu.make_async_copy(q_hbm.at[sch.at[rs]], qrow.at[rs], gsems[r]),
                    pltpu.make_async_copy(h_hbm.at[sch.at[rs]], hrows.at[rs], gsems[r]))

        for d in edge_descs(0):
            d.start()

        @pl.loop(0, nchunk)
        def _chunk(c):
            for d in edge_descs(c):
                d.wait()
            for r in range(nsub):
                for k in offs:
                    didxs[r][pl.ds(k, 16)] = dch[pl.ds(r * sub + k, 16)]
            for r in range(nsub):
                for d in gather_descs(r):
                    d.start()

            @pl.when(c + 1 < nchunk)
            def _():
                for d in edge_descs(c + 1):
                    d.start()

            sdescs = []
            for r in range(nsub):
                mb, ssem = (msg0, ssem0) if r % 2 == 0 else (msg1, ssem1)
                for d in gather_descs(r):
                    d.wait()
                if r >= 2:
                    sdescs[r - 2].wait()

                @pl.loop(0, sub, unroll=4)
                def _e(e, _r=r, _mb=mb):
                    ge = _r * sub + e
                    t = prow[ge, :] + qrow[ge, :]
                    u = jnp.where(t >= 0, t, t * NEG_SLOPE)
                    if nheads == 8:
                        lu = t.at[sh_idx].get(mode="promise_in_bounds")
                        a16 = jnp.exp(u + lu)
                        for h in range(8):
                            ah = a16.at[lane * 0 + h].get(mode="promise_in_bounds")
                            _mb[e, pl.ds(h * 16, 16)] = ah * hrows[ge, pl.ds(h * 16, 16)]
                    else:
                        a0 = u.at[lane * 0].get(mode="promise_in_bounds")
                        l0 = t.at[lane * 0 + 1].get(mode="promise_in_bounds")
                        a16 = jnp.exp(a0 + l0)
                        for qd in range(fdim // 16):
                            _mb[e, pl.ds(qd * 16, 16)] = a16 * hrows[ge, pl.ds(qd * 16, 16)]

                sd = pltpu.make_async_copy(mb, acc.at[didxs[r]], ssem)
                sd.start(add=True)
                sdescs.append(sd)
            sdescs[nsub - 2].wait()
            sdescs[nsub - 1].wait()

        plsc.subcore_barrier()
        pltpu.sync_copy(acc.at[pl.ds(s0, STRIPE)], out_hbm.at[cid, pl.ds(s0, STRIPE)])

    return pl.kernel(
        body,
        out_type=jax.ShapeDtypeStruct((NC, N_NODES, fdim), jnp.float32),
        mesh=_MESH,
        compiler_params=_SC_PARAMS,
        scratch_types=[
            pltpu.VMEM((ch,), jnp.int32),
            pltpu.VMEM((ch,), jnp.int32),
            pltpu.VMEM((sub,), jnp.int32),
            pltpu.VMEM((sub,), jnp.int32),
            pltpu.VMEM((sub,), jnp.int32),
            pltpu.VMEM((sub,), jnp.int32),
            pltpu.VMEM((sub,), jnp.int32),
            pltpu.VMEM((ch, 16), jnp.float32),
            pltpu.VMEM((ch, 16), jnp.float32),
            pltpu.VMEM((ch, fdim), jnp.float32),
            pltpu.VMEM((sub, fdim), jnp.float32),
            pltpu.VMEM((sub, fdim), jnp.float32),
            pltpu.VMEM_SHARED((N_NODES, fdim), jnp.float32),
        ] + [pltpu.SemaphoreType.DMA] * 8,
    )


_sc_msg1 = _make_sc_msg(HEADS * HID, HEADS, 200, 40)
_sc_msg2 = _make_sc_msg(OUT_DIM, 1, 400, 80)


def _fin_body(p_ref, out_ref):
    o = p_ref[0] + p_ref[1]
    m = jnp.max(o, axis=1, keepdims=True)
    l = o - m
    out_ref[...] = l - jnp.log(jnp.sum(jnp.exp(l), axis=1, keepdims=True))


def _fin(parts):
    n, d = parts.shape[1], parts.shape[2]
    blk = 2000
    return pl.pallas_call(
        _fin_body,
        grid=(n // blk,),
        in_specs=[pl.BlockSpec((2, blk, d), lambda i: (0, i, 0))],
        out_specs=pl.BlockSpec((blk, d), lambda i: (i, 0)),
        out_shape=jax.ShapeDtypeStruct((n, d), jnp.float32),
    )(parts)


def kernel(x, edge_index, W1, att_src1, att_dst1, W2, att_src2, att_dst2):
    src = edge_index[0].astype(jnp.int32)
    dst = edge_index[1].astype(jnp.int32)

    A1 = jnp.concatenate([_block_diag_att(att_src1), _block_diag_att(att_dst1)], axis=1)
    h1, sd1, mx1 = _tc1(x, W1, A1)
    b1 = jax.nn.leaky_relu(mx1[0, :HEADS] + mx1[0, HEADS:], NEG_SLOPE)
    b_rows1 = jnp.broadcast_to(b1[:, None], (HEADS, 16))
    den1 = _sc_denom1(src, dst, sd1.T, b_rows1)
    bp1 = jnp.concatenate([jnp.zeros((8,), jnp.float32), b1])[None, :]
    P1 = _p1(den1, sd1, bp1)
    Q1 = jnp.concatenate([sd1[:, HEADS:], jnp.zeros((N_NODES, 8), jnp.float32)], axis=1)
    z1 = jnp.zeros((STRIPE, HEADS * HID), jnp.float32)
    out1 = _sc_msg1(src, dst, P1, Q1, h1, z1)

    A2 = jnp.concatenate([_block_diag_att(att_src2, pad_to=8),
                          _block_diag_att(att_dst2, pad_to=8)], axis=1)
    h2, sd2, mx2 = _tc2(out1, W2, A2)
    b2 = jax.nn.leaky_relu(mx2[0, 0] + mx2[0, 8], NEG_SLOPE)
    b_rows2 = jnp.broadcast_to(b2[None, None], (1, 16))
    den2 = _sc_denom2(src, dst, sd2.T, b_rows2)
    bp2 = jnp.zeros((1, 16), jnp.float32).at[0, 1].set(b2)
    P2 = _p2(den2, sd2, bp2)
    Q2 = jnp.concatenate([sd2[:, 8:9], jnp.zeros((N_NODES, 15), jnp.float32)], axis=1)
    z2 = jnp.zeros((STRIPE, OUT_DIM), jnp.float32)
    out2 = _sc_msg2(src, dst, P2, Q2, h2, z2)

    return _fin(out2)


# speedup vs baseline: 40.6518x; 1.0256x over previous
import functools

import jax
import jax.numpy as jnp
from jax import lax
from jax.experimental import pallas as pl
from jax.experimental.pallas import tpu as pltpu, tpu_sc as plsc

N_NODES = 10000
N_EDGES = 320000
IN_DIM = 128
HID = 16
HEADS = 8
OUT_DIM = 64
NEG_SLOPE = 0.2

NC = 2
NS = 16
NW = NC * NS
EPW = N_EDGES // NW
CH = 200
NCHUNK = EPW // CH
STRIPE = 632

_MESH = plsc.VectorSubcoreMesh(core_axis_name="c", subcore_axis_name="s",
                               num_cores=NC, num_subcores=NS)
_SC_PARAMS = pltpu.CompilerParams(needs_layout_passes=False,
                                  use_tc_tiling_on_sc=False)


def _block_diag_att(att, pad_to=None):
    _, H, C = att.shape
    a = att.reshape(H, C)
    eye = jnp.eye(H, dtype=att.dtype)
    out = (a[:, :, None] * eye[:, None, :]).reshape(H * C, H)
    if pad_to is not None and pad_to > H:
        out = jnp.concatenate([out, jnp.zeros((H * C, pad_to - H), att.dtype)], axis=1)
    return out


def _tc1_body(x_ref, w_ref, a_ref, h_ref, sd_ref, mx_ref):
    h = jnp.dot(x_ref[...], w_ref[...], preferred_element_type=jnp.float32)
    h_ref[...] = h
    sd = jnp.dot(h, a_ref[...], preferred_element_type=jnp.float32)
    sd_ref[...] = sd
    bmax = jnp.max(sd, axis=0, keepdims=True)

    @pl.when(pl.program_id(0) == 0)
    def _():
        mx_ref[...] = bmax

    @pl.when(pl.program_id(0) != 0)
    def _():
        mx_ref[...] = jnp.maximum(mx_ref[...], bmax)


def _tc1(x, W, A):
    n, k = x.shape
    m = W.shape[1]
    blk = 400
    return pl.pallas_call(
        _tc1_body,
        grid=(n // blk,),
        in_specs=[
            pl.BlockSpec((blk, k), lambda i: (i, 0)),
            pl.BlockSpec((k, m), lambda i: (0, 0)),
            pl.BlockSpec((m, 16), lambda i: (0, 0)),
        ],
        out_specs=[
            pl.BlockSpec((blk, m), lambda i: (i, 0)),
            pl.BlockSpec((blk, 16), lambda i: (i, 0)),
            pl.BlockSpec((1, 16), lambda i: (0, 0)),
        ],
        out_shape=[
            jax.ShapeDtypeStruct((n, m), jnp.float32),
            jax.ShapeDtypeStruct((n, 16), jnp.float32),
            jax.ShapeDtypeStruct((1, 16), jnp.float32),
        ],
    )(x, W, A)


def _tc2_body(p_ref, w_ref, a_ref, h_ref, sd_ref, mx_ref):
    hin = p_ref[0] + p_ref[1]
    hin = jnp.where(hin > 0, hin, jnp.exp(jnp.minimum(hin, 0.0)) - 1.0)
    h = jnp.dot(hin, w_ref[...], preferred_element_type=jnp.float32)
    h_ref[...] = h
    sd = jnp.dot(h, a_ref[...], preferred_element_type=jnp.float32)
    sd_ref[...] = sd
    bmax = jnp.max(sd, axis=0, keepdims=True)

    @pl.when(pl.program_id(0) == 0)
    def _():
        mx_ref[...] = bmax

    @pl.when(pl.program_id(0) != 0)
    def _():
        mx_ref[...] = jnp.maximum(mx_ref[...], bmax)


def _tc2(parts, W, A):
    n = parts.shape[1]
    k = parts.shape[2]
    m = W.shape[1]
    blk = 400
    return pl.pallas_call(
        _tc2_body,
        grid=(n // blk,),
        in_specs=[
            pl.BlockSpec((2, blk, k), lambda i: (0, i, 0)),
            pl.BlockSpec((k, m), lambda i: (0, 0)),
            pl.BlockSpec((m, 16), lambda i: (0, 0)),
        ],
        out_specs=[
            pl.BlockSpec((blk, m), lambda i: (i, 0)),
            pl.BlockSpec((blk, 16), lambda i: (i, 0)),
            pl.BlockSpec((1, 16), lambda i: (0, 0)),
        ],
        out_shape=[
            jax.ShapeDtypeStruct((n, m), jnp.float32),
            jax.ShapeDtypeStruct((n, 16), jnp.float32),
            jax.ShapeDtypeStruct((1, 16), jnp.float32),
        ],
    )(parts, W, A)


def _p1_body(den_ref, sd_ref, bp_ref, p_ref):
    d8 = jnp.sum(den_ref[...], axis=0)
    r = -jnp.log(d8 + 1e-16)
    p_ref[...] = jnp.concatenate([sd_ref[:, :8], r.T], axis=1) - bp_ref[...]


def _p1(den_parts, sd, bp):
    n = sd.shape[0]
    return pl.pallas_call(
        _p1_body,
        out_shape=jax.ShapeDtypeStruct((n, 16), jnp.float32),
    )(den_parts, sd, bp)


def _p2_body(den_ref, sd_ref, bp_ref, p_ref):
    d = jnp.sum(den_ref[...], axis=0)
    r = -jnp.log(d + 1e-16)
    bn = d.shape[0]
    p = jnp.concatenate(
        [sd_ref[:, 0:1], r[:, None], jnp.zeros((bn, 14), jnp.float32)], axis=1)
    p_ref[...] = p - bp_ref[...]


def _p2(den_parts, sd, bp):
    n = sd.shape[0]
    return pl.pallas_call(
        _p2_body,
        out_shape=jax.ShapeDtypeStruct((n, 16), jnp.float32),
    )(den_parts, sd, bp)


@functools.partial(
    pl.kernel,
    out_type=jax.ShapeDtypeStruct((NW, HEADS, N_NODES), jnp.float32),
    mesh=_MESH,
    compiler_params=_SC_PARAMS,
    scratch_types=[
        pltpu.VMEM((EPW,), jnp.int32),
        pltpu.VMEM((EPW,), jnp.int32),
        pltpu.VMEM((2, N_NODES), jnp.float32),
        pltpu.VMEM((2, N_NODES), jnp.float32),
        pltpu.VMEM((N_NODES,), jnp.float32),
        pltpu.VMEM((HEADS, 16), jnp.float32),
        pltpu.SemaphoreType.DMA,
        pltpu.SemaphoreType.DMA,
    ],
)
def _sc_denom1(src_hbm, dst_hbm, sdT_hbm, b_hbm, zn_hbm, den_out,
               src_v, dst_v, s_tab, d_tab, den_tab, b_v, tsem0, tsem1):
    cid = lax.axis_index("c")
    sid = lax.axis_index("s")
    wid = sid * NC + cid
    base = wid * EPW
    tsems = [tsem0, tsem1]

    def tab_descs(h, b):
        return (pltpu.make_async_copy(sdT_hbm.at[h], s_tab.at[b], tsems[b]),
                pltpu.make_async_copy(sdT_hbm.at[h + HEADS], d_tab.at[b], tsems[b]))

    for d in tab_descs(0, 0):
        d.start()
    pltpu.sync_copy(b_hbm, b_v)
    pltpu.sync_copy(src_hbm.at[pl.ds(base, EPW)], src_v)
    pltpu.sync_copy(dst_hbm.at[pl.ds(base, EPW)], dst_v)

    for h in range(HEADS):
        b = h % 2
        if h + 1 < HEADS:
            for d in tab_descs(h + 1, 1 - b):
                d.start()
        pltpu.sync_copy(zn_hbm, den_tab)
        for d in tab_descs(h, b):
            d.wait()
        bh = b_v[h, :]

        @pl.loop(0, EPW // 16, unroll=4)
        def _g(g, _b=b, _bh=bh):
            di = dst_v[pl.ds(g * 16, 16)]
            sj = src_v[pl.ds(g * 16, 16)]
            t = plsc.load_gather(s_tab.at[_b], [di]) + plsc.load_gather(d_tab.at[_b], [sj])
            al = jnp.where(t >= 0, t, t * NEG_SLOPE)
            e = jnp.exp(al - _bh)
            plsc.addupdate_scatter(den_tab, [di], e)

        pltpu.sync_copy(den_tab, den_out.at[wid, h])


@functools.partial(
    pl.kernel,
    out_type=jax.ShapeDtypeStruct((NW, N_NODES), jnp.float32),
    mesh=_MESH,
    compiler_params=_SC_PARAMS,
    scratch_types=[
        pltpu.VMEM((EPW,), jnp.int32),
        pltpu.VMEM((EPW,), jnp.int32),
        pltpu.VMEM((N_NODES,), jnp.float32),
        pltpu.VMEM((N_NODES,), jnp.float32),
        pltpu.VMEM((N_NODES,), jnp.float32),
        pltpu.VMEM((16,), jnp.float32),
    ],
)
def _sc_denom2(src_hbm, dst_hbm, sdT_hbm, b_hbm, zn_hbm, den_out,
               src_v, dst_v, s_tab, d_tab, den_tab, b_v):
    cid = lax.axis_index("c")
    sid = lax.axis_index("s")
    wid = sid * NC + cid
    base = wid * EPW
    pltpu.sync_copy(src_hbm.at[pl.ds(base, EPW)], src_v)
    pltpu.sync_copy(dst_hbm.at[pl.ds(base, EPW)], dst_v)
    pltpu.sync_copy(sdT_hbm.at[0], s_tab)
    pltpu.sync_copy(sdT_hbm.at[8], d_tab)
    pltpu.sync_copy(b_hbm.at[0], b_v)
    pltpu.sync_copy(zn_hbm, den_tab)
    bh = b_v[...]

    @pl.loop(0, EPW // 16, unroll=4)
    def _g(g):
        di = dst_v[pl.ds(g * 16, 16)]
        sj = src_v[pl.ds(g * 16, 16)]
        t = plsc.load_gather(s_tab, [di]) + plsc.load_gather(d_tab, [sj])
        al = jnp.where(t >= 0, t, t * NEG_SLOPE)
        e = jnp.exp(al - bh)
        plsc.addupdate_scatter(den_tab, [di], e)

    pltpu.sync_copy(den_tab, den_out.at[wid])


def _make_sc_msg(fdim, nheads, ch, sub):
    nsub = ch // sub
    nchunk = EPW // ch
    assert nsub == 5 and ch % sub == 0 and EPW % ch == 0
    offs = sorted(set(list(range(0, sub - 15, 16)) + [sub - 16]))

    def body(src_hbm, dst_hbm, p_hbm, q_hbm, h_hbm, z_hbm, out_hbm,
             sch, dch, didx0, didx1, didx2, didx3, didx4, prow, qrow, hrows,
             msg0, msg1, acc, esem, g0, g1, g2, g3, g4, ssem0, ssem1):
        cid = lax.axis_index("c")
        sid = lax.axis_index("s")
        wid = sid * NC + cid
        base = wid * EPW
        s0 = jnp.minimum(sid * STRIPE, N_NODES - STRIPE)
        pltpu.sync_copy(z_hbm, acc.at[pl.ds(s0, STRIPE)])
        plsc.subcore_barrier()

        didxs = [didx0, didx1, didx2, didx3, didx4]
        gsems = [g0, g1, g2, g3, g4]
        lane = lax.iota(jnp.int32, 16)
        sh_idx = (lane & 7) + 8

        def edge_descs(c):
            return (pltpu.make_async_copy(src_hbm.at[pl.ds(base + c * ch, ch)], sch, esem),
                    pltpu.make_async_copy(dst_hbm.at[pl.ds(base + c * ch, ch)], dch, esem))

        def gather_descs(r):
            rs = pl.ds(r * sub, sub)
            return (pltpu.make_async_copy(p_hbm.at[didxs[r]], prow.at[rs], gsems[r]),
                    pltpu.make_async_copy(q_hbm.at[sch.at[rs]], qrow.at[rs], gsems[r]),
                    pltpu.make_async_copy(h_hbm.at[sch.at[rs]], hrows.at[rs], gsems[r]))

        for d in edge_descs(0):
            d.start()

        @pl.loop(0, nchunk)
        def _chunk(c):
            for d in edge_descs(c):
                d.wait()
            for r in range(nsub):
                for k in offs:
                    didxs[r][pl.ds(k, 16)] = dch[pl.ds(r * sub + k, 16)]
            for r in range(nsub):
                for d in gather_descs(r):
                    d.start()

            @pl.when(c + 1 < nchunk)
            def _():
                for d in edge_descs(c + 1):
                    d.start()

            sdescs = []
            for r in range(nsub):
                mb, ssem = (msg0, ssem0) if r % 2 == 0 else (msg1, ssem1)
                for d in gather_descs(r):
                    d.wait()
                if r >= 2:
                    sdescs[r - 2].wait()

                @pl.loop(0, sub, unroll=4)
                def _e(e, _r=r, _mb=mb):
                    ge = _r * sub + e
                    t = prow[ge, :] + qrow[ge, :]
                    u = jnp.where(t >= 0, t, t * NEG_SLOPE)
                    if nheads == 8:
                        lu = t.at[sh_idx].get(mode="promise_in_bounds")
                        a16 = jnp.exp(u + lu)
                        for h in range(8):
                            ah = a16.at[lane * 0 + h].get(mode="promise_in_bounds")
                            _mb[e, pl.ds(h * 16, 16)] = ah * hrows[ge, pl.ds(h * 16, 16)]
                    else:
                        a0 = u.at[lane * 0].get(mode="promise_in_bounds")
                        l0 = t.at[lane * 0 + 1].get(mode="promise_in_bounds")
                        a16 = jnp.exp(a0 + l0)
                        for qd in range(fdim // 16):
                            _mb[e, pl.ds(qd * 16, 16)] = a16 * hrows[ge, pl.ds(qd * 16, 16)]

                sd = pltpu.make_async_copy(mb, acc.at[didxs[r]], ssem)
                sd.start(add=True)
                sdescs.append(sd)
            sdescs[nsub - 2].wait()
            sdescs[nsub - 1].wait()

        plsc.subcore_barrier()
        pltpu.sync_copy(acc.at[pl.ds(s0, STRIPE)], out_hbm.at[cid, pl.ds(s0, STRIPE)])

    return pl.kernel(
        body,
        out_type=jax.ShapeDtypeStruct((NC, N_NODES, fdim), jnp.float32),
        mesh=_MESH,
        compiler_params=_SC_PARAMS,
        scratch_types=[
            pltpu.VMEM((ch,), jnp.int32),
            pltpu.VMEM((ch,), jnp.int32),
            pltpu.VMEM((sub,), jnp.int32),
            pltpu.VMEM((sub,), jnp.int32),
            pltpu.VMEM((sub,), jnp.int32),
            pltpu.VMEM((sub,), jnp.int32),
            pltpu.VMEM((sub,), jnp.int32),
            pltpu.VMEM((ch, 16), jnp.float32),
            pltpu.VMEM((ch, 16), jnp.float32),
            pltpu.VMEM((ch, fdim), jnp.float32),
            pltpu.VMEM((sub, fdim), jnp.float32),
            pltpu.VMEM((sub, fdim), jnp.float32),
            pltpu.VMEM_SHARED((N_NODES, fdim), jnp.float32),
        ] + [pltpu.SemaphoreType.DMA] * 8,
    )


_sc_msg1 = _make_sc_msg(HEADS * HID, HEADS, 200, 40)
_sc_msg2 = _make_sc_msg(OUT_DIM, 1, 400, 80)


def _fin_body(p_ref, out_ref):
    o = p_ref[0] + p_ref[1]
    m = jnp.max(o, axis=1, keepdims=True)
    l = o - m
    out_ref[...] = l - jnp.log(jnp.sum(jnp.exp(l), axis=1, keepdims=True))


def _fin(parts):
    n, d = parts.shape[1], parts.shape[2]
    blk = 2000
    return pl.pallas_call(
        _fin_body,
        grid=(n // blk,),
        in_specs=[pl.BlockSpec((2, blk, d), lambda i: (0, i, 0))],
        out_specs=pl.BlockSpec((blk, d), lambda i: (i, 0)),
        out_shape=jax.ShapeDtypeStruct((n, d), jnp.float32),
    )(parts)


def kernel(x, edge_index, W1, att_src1, att_dst1, W2, att_src2, att_dst2):
    src = edge_index[0].astype(jnp.int32)
    dst = edge_index[1].astype(jnp.int32)

    A1 = jnp.concatenate([_block_diag_att(att_src1), _block_diag_att(att_dst1)], axis=1)
    h1, sd1, mx1 = _tc1(x, W1, A1)
    b1 = jax.nn.leaky_relu(mx1[0, :HEADS] + mx1[0, HEADS:], NEG_SLOPE)
    b_rows1 = jnp.broadcast_to(b1[:, None], (HEADS, 16))
    zn = jnp.zeros((N_NODES,), jnp.float32)
    den1 = _sc_denom1(src, dst, sd1.T, b_rows1, zn)
    bp1 = jnp.concatenate([jnp.zeros((8,), jnp.float32), b1])[None, :]
    P1 = _p1(den1, sd1, bp1)
    Q1 = jnp.concatenate([sd1[:, HEADS:], jnp.zeros((N_NODES, 8), jnp.float32)], axis=1)
    z1 = jnp.zeros((STRIPE, HEADS * HID), jnp.float32)
    out1 = _sc_msg1(src, dst, P1, Q1, h1, z1)

    A2 = jnp.concatenate([_block_diag_att(att_src2, pad_to=8),
                          _block_diag_att(att_dst2, pad_to=8)], axis=1)
    h2, sd2, mx2 = _tc2(out1, W2, A2)
    b2 = jax.nn.leaky_relu(mx2[0, 0] + mx2[0, 8], NEG_SLOPE)
    b_rows2 = jnp.broadcast_to(b2[None, None], (1, 16))
    den2 = _sc_denom2(src, dst, sd2.T, b_rows2, zn)
    bp2 = jnp.zeros((1, 16), jnp.float32).at[0, 1].set(b2)
    P2 = _p2(den2, sd2, bp2)
    Q2 = jnp.concatenate([sd2[:, 8:9], jnp.zeros((N_NODES, 15), jnp.float32)], axis=1)
    z2 = jnp.zeros((STRIPE, OUT_DIM), jnp.float32)
    out2 = _sc_msg2(src, dst, P2, Q2, h2, z2)

    return _fin(out2)


# speedup vs baseline: 80.6115x; 1.9830x over previous
import functools

import jax
import jax.numpy as jnp
from jax import lax
from jax.experimental import pallas as pl
from jax.experimental.pallas import tpu as pltpu, tpu_sc as plsc

N_NODES = 10000
N_EDGES = 320000
IN_DIM = 128
HID = 16
HEADS = 8
OUT_DIM = 64
NEG_SLOPE = 0.2

NC = 2
NS = 16
NW = NC * NS
EPW = N_EDGES // NW
CH = 200
NCHUNK = EPW // CH
STRIPE = 632

_MESH = plsc.VectorSubcoreMesh(core_axis_name="c", subcore_axis_name="s",
                               num_cores=NC, num_subcores=NS)
_SC_PARAMS = pltpu.CompilerParams(needs_layout_passes=False,
                                  use_tc_tiling_on_sc=False)


def _block_diag_att(att, pad_to=None):
    _, H, C = att.shape
    a = att.reshape(H, C)
    eye = jnp.eye(H, dtype=att.dtype)
    out = (a[:, :, None] * eye[:, None, :]).reshape(H * C, H)
    if pad_to is not None and pad_to > H:
        out = jnp.concatenate([out, jnp.zeros((H * C, pad_to - H), att.dtype)], axis=1)
    return out


def _tc1_body(x_ref, w_ref, a_ref, h_ref, sd_ref, mx_ref):
    h = jnp.dot(x_ref[...], w_ref[...], preferred_element_type=jnp.float32)
    h_ref[...] = h
    sd = jnp.dot(h, a_ref[...], preferred_element_type=jnp.float32)
    sd_ref[...] = sd
    bmax = jnp.max(sd, axis=0, keepdims=True)

    @pl.when(pl.program_id(0) == 0)
    def _():
        mx_ref[...] = bmax

    @pl.when(pl.program_id(0) != 0)
    def _():
        mx_ref[...] = jnp.maximum(mx_ref[...], bmax)


def _tc1(x, W, A):
    n, k = x.shape
    m = W.shape[1]
    blk = 400
    return pl.pallas_call(
        _tc1_body,
        grid=(n // blk,),
        in_specs=[
            pl.BlockSpec((blk, k), lambda i: (i, 0)),
            pl.BlockSpec((k, m), lambda i: (0, 0)),
            pl.BlockSpec((m, 16), lambda i: (0, 0)),
        ],
        out_specs=[
            pl.BlockSpec((blk, m), lambda i: (i, 0)),
            pl.BlockSpec((blk, 16), lambda i: (i, 0)),
            pl.BlockSpec((1, 16), lambda i: (0, 0)),
        ],
        out_shape=[
            jax.ShapeDtypeStruct((n, m), jnp.float32),
            jax.ShapeDtypeStruct((n, 16), jnp.float32),
            jax.ShapeDtypeStruct((1, 16), jnp.float32),
        ],
    )(x, W, A)


def _tc2_body(p_ref, w_ref, a_ref, h_ref, sd_ref, mx_ref):
    hin = p_ref[0] + p_ref[1]
    hin = jnp.where(hin > 0, hin, jnp.exp(jnp.minimum(hin, 0.0)) - 1.0)
    h = jnp.dot(hin, w_ref[...], preferred_element_type=jnp.float32)
    h_ref[...] = h
    sd = jnp.dot(h, a_ref[...], preferred_element_type=jnp.float32)
    sd_ref[...] = sd
    bmax = jnp.max(sd, axis=0, keepdims=True)

    @pl.when(pl.program_id(0) == 0)
    def _():
        mx_ref[...] = bmax

    @pl.when(pl.program_id(0) != 0)
    def _():
        mx_ref[...] = jnp.maximum(mx_ref[...], bmax)


def _tc2(parts, W, A):
    n = parts.shape[1]
    k = parts.shape[2]
    m = W.shape[1]
    blk = 400
    return pl.pallas_call(
        _tc2_body,
        grid=(n // blk,),
        in_specs=[
            pl.BlockSpec((2, blk, k), lambda i: (0, i, 0)),
            pl.BlockSpec((k, m), lambda i: (0, 0)),
            pl.BlockSpec((m, 16), lambda i: (0, 0)),
        ],
        out_specs=[
            pl.BlockSpec((blk, m), lambda i: (i, 0)),
            pl.BlockSpec((blk, 16), lambda i: (i, 0)),
            pl.BlockSpec((1, 16), lambda i: (0, 0)),
        ],
        out_shape=[
            jax.ShapeDtypeStruct((n, m), jnp.float32),
            jax.ShapeDtypeStruct((n, 16), jnp.float32),
            jax.ShapeDtypeStruct((1, 16), jnp.float32),
        ],
    )(parts, W, A)


def _p1_body(den_ref, sd_ref, bp_ref, p_ref):
    d8 = jnp.sum(den_ref[...], axis=0)
    r = -jnp.log(d8 + 1e-16)
    p_ref[...] = jnp.concatenate([sd_ref[:, :8], r.T], axis=1) - bp_ref[...]


def _p1(den_parts, sd, bp):
    n = sd.shape[0]
    return pl.pallas_call(
        _p1_body,
        out_shape=jax.ShapeDtypeStruct((n, 16), jnp.float32),
    )(den_parts, sd, bp)


def _p2_body(den_ref, sd_ref, bp_ref, p_ref):
    d = jnp.sum(den_ref[...], axis=0)
    r = -jnp.log(d + 1e-16)
    bn = d.shape[0]
    p = jnp.concatenate(
        [sd_ref[:, 0:1], r[:, None], jnp.zeros((bn, 14), jnp.float32)], axis=1)
    p_ref[...] = p - bp_ref[...]


def _p2(den_parts, sd, bp):
    n = sd.shape[0]
    return pl.pallas_call(
        _p2_body,
        out_shape=jax.ShapeDtypeStruct((n, 16), jnp.float32),
    )(den_parts, sd, bp)


@functools.partial(
    pl.kernel,
    out_type=jax.ShapeDtypeStruct((NW, HEADS, N_NODES), jnp.float32),
    mesh=_MESH,
    compiler_params=_SC_PARAMS,
    scratch_types=[
        pltpu.VMEM((EPW,), jnp.int32),
        pltpu.VMEM((EPW,), jnp.int32),
        pltpu.VMEM((2, N_NODES), jnp.float32),
        pltpu.VMEM((2, N_NODES), jnp.float32),
        pltpu.VMEM((N_NODES,), jnp.float32),
        pltpu.VMEM((HEADS, 16), jnp.float32),
        pltpu.SemaphoreType.DMA,
        pltpu.SemaphoreType.DMA,
    ],
)
def _sc_denom1(src_hbm, dst_hbm, sdT_hbm, b_hbm, zn_hbm, den_out,
               src_v, dst_v, s_tab, d_tab, den_tab, b_v, tsem0, tsem1):
    cid = lax.axis_index("c")
    sid = lax.axis_index("s")
    wid = sid * NC + cid
    base = wid * EPW
    tsems = [tsem0, tsem1]

    def tab_descs(h, b):
        return (pltpu.make_async_copy(sdT_hbm.at[h], s_tab.at[b], tsems[b]),
                pltpu.make_async_copy(sdT_hbm.at[h + HEADS], d_tab.at[b], tsems[b]))

    for d in tab_descs(0, 0):
        d.start()
    pltpu.sync_copy(b_hbm, b_v)
    pltpu.sync_copy(src_hbm.at[pl.ds(base, EPW)], src_v)
    pltpu.sync_copy(dst_hbm.at[pl.ds(base, EPW)], dst_v)

    for h in range(HEADS):
        b = h % 2
        if h + 1 < HEADS:
            for d in tab_descs(h + 1, 1 - b):
                d.start()
        pltpu.sync_copy(zn_hbm, den_tab)
        for d in tab_descs(h, b):
            d.wait()
        bh = b_v[h, :]

        @pl.loop(0, EPW // 16, unroll=4)
        def _g(g, _b=b, _bh=bh):
            di = dst_v[pl.ds(g * 16, 16)]
            sj = src_v[pl.ds(g * 16, 16)]
            t = plsc.load_gather(s_tab.at[_b], [di]) + plsc.load_gather(d_tab.at[_b], [sj])
            al = jnp.where(t >= 0, t, t * NEG_SLOPE)
            e = jnp.exp(al - _bh)
            plsc.addupdate_scatter(den_tab, [di], e)

        pltpu.sync_copy(den_tab, den_out.at[wid, h])


@functools.partial(
    pl.kernel,
    out_type=jax.ShapeDtypeStruct((NW, N_NODES), jnp.float32),
    mesh=_MESH,
    compiler_params=_SC_PARAMS,
    scratch_types=[
        pltpu.VMEM((EPW,), jnp.int32),
        pltpu.VMEM((EPW,), jnp.int32),
        pltpu.VMEM((N_NODES,), jnp.float32),
        pltpu.VMEM((N_NODES,), jnp.float32),
        pltpu.VMEM((N_NODES,), jnp.float32),
        pltpu.VMEM((16,), jnp.float32),
    ],
)
def _sc_denom2(src_hbm, dst_hbm, sdT_hbm, b_hbm, zn_hbm, den_out,
               src_v, dst_v, s_tab, d_tab, den_tab, b_v):
    cid = lax.axis_index("c")
    sid = lax.axis_index("s")
    wid = sid * NC + cid
    base = wid * EPW
    pltpu.sync_copy(src_hbm.at[pl.ds(base, EPW)], src_v)
    pltpu.sync_copy(dst_hbm.at[pl.ds(base, EPW)], dst_v)
    pltpu.sync_copy(sdT_hbm.at[0], s_tab)
    pltpu.sync_copy(sdT_hbm.at[8], d_tab)
    pltpu.sync_copy(b_hbm.at[0], b_v)
    pltpu.sync_copy(zn_hbm, den_tab)
    bh = b_v[...]

    @pl.loop(0, EPW // 16, unroll=4)
    def _g(g):
        di = dst_v[pl.ds(g * 16, 16)]
        sj = src_v[pl.ds(g * 16, 16)]
        t = plsc.load_gather(s_tab, [di]) + plsc.load_gather(d_tab, [sj])
        al = jnp.where(t >= 0, t, t * NEG_SLOPE)
        e = jnp.exp(al - bh)
        plsc.addupdate_scatter(den_tab, [di], e)

    pltpu.sync_copy(den_tab, den_out.at[wid])


def _make_sc_msg(fdim, nheads, ch, sub):
    nsub = ch // sub
    nchunk = EPW // ch
    assert nsub == 5 and ch % sub == 0 and EPW % ch == 0
    offs = sorted(set(list(range(0, sub - 15, 16)) + [sub - 16]))

    def body(src_hbm, dst_hbm, p_hbm, q_hbm, h_hbm, z_hbm, out_hbm,
             sch, dch, didx0, didx1, didx2, didx3, didx4, prow, qrow, hrows,
             msg0, msg1, acc, esem, g0, g1, g2, g3, g4, ssem0, ssem1):
        cid = lax.axis_index("c")
        sid = lax.axis_index("s")
        wid = sid * NC + cid
        base = wid * EPW
        s0 = jnp.minimum(sid * STRIPE, N_NODES - STRIPE)
        pltpu.sync_copy(z_hbm, acc.at[pl.ds(s0, STRIPE)])
        plsc.subcore_barrier()

        didxs = [didx0, didx1, didx2, didx3, didx4]
        gsems = [g0, g1, g2, g3, g4]
        lane = lax.iota(jnp.int32, 16)
        sh_idx = (lane & 7) + 8

        def edge_descs(c):
            return (pltpu.make_async_copy(src_hbm.at[pl.ds(base + c * ch, ch)], sch, esem),
                    pltpu.make_async_copy(dst_hbm.at[pl.ds(base + c * ch, ch)], dch, esem))

        def gather_descs(r):
            rs = pl.ds(r * sub, sub)
            return (pltpu.make_async_copy(p_hbm.at[didxs[r]], prow.at[rs], gsems[r]),
                    pltpu.make_async_copy(q_hbm.at[sch.at[rs]], qrow.at[rs], gsems[r]),
                    pltpu.make_async_copy(h_hbm.at[sch.at[rs]], hrows.at[rs], gsems[r]))

        for d in edge_descs(0):
            d.start()

        @pl.loop(0, nchunk)
        def _chunk(c):
            for d in edge_descs(c):
                d.wait()
            for r in range(nsub):
                for k in offs:
                    didxs[r][pl.ds(k, 16)] = dch[pl.ds(r * sub + k, 16)]
            for r in range(nsub):
                for d in gather_descs(r):
                    d.start()

            @pl.when(c + 1 < nchunk)
            def _():
                for d in edge_descs(c + 1):
                    d.start()

            sdescs = []
            for r in range(nsub):
                mb, ssem = (msg0, ssem0) if r % 2 == 0 else (msg1, ssem1)
                for d in gather_descs(r):
                    d.wait()
                if r >= 2:
                    sdescs[r - 2].wait()

                @plsc.parallel_loop(0, sub, unroll=4)
                def _e(e, _r=r, _mb=mb):
                    ge = _r * sub + e
                    t = prow[ge, :] + qrow[ge, :]
                    u = jnp.where(t >= 0, t, t * NEG_SLOPE)
                    if nheads == 8:
                        lu = t.at[sh_idx].get(mode="promise_in_bounds")
                        a16 = jnp.exp(u + lu)
                        for h in range(8):
                            ah = a16.at[lane * 0 + h].get(mode="promise_in_bounds")
                            _mb[e, pl.ds(h * 16, 16)] = ah * hrows[ge, pl.ds(h * 16, 16)]
                    else:
                        a0 = u.at[lane * 0].get(mode="promise_in_bounds")
                        l0 = t.at[lane * 0 + 1].get(mode="promise_in_bounds")
                        a16 = jnp.exp(a0 + l0)
                        for qd in range(fdim // 16):
                            _mb[e, pl.ds(qd * 16, 16)] = a16 * hrows[ge, pl.ds(qd * 16, 16)]

                sd = pltpu.make_async_copy(mb, acc.at[didxs[r]], ssem)
                sd.start(add=True)
                sdescs.append(sd)
            sdescs[nsub - 2].wait()
            sdescs[nsub - 1].wait()

        plsc.subcore_barrier()
        pltpu.sync_copy(acc.at[pl.ds(s0, STRIPE)], out_hbm.at[cid, pl.ds(s0, STRIPE)])

    return pl.kernel(
        body,
        out_type=jax.ShapeDtypeStruct((NC, N_NODES, fdim), jnp.float32),
        mesh=_MESH,
        compiler_params=_SC_PARAMS,
        scratch_types=[
            pltpu.VMEM((ch,), jnp.int32),
            pltpu.VMEM((ch,), jnp.int32),
            pltpu.VMEM((sub,), jnp.int32),
            pltpu.VMEM((sub,), jnp.int32),
            pltpu.VMEM((sub,), jnp.int32),
            pltpu.VMEM((sub,), jnp.int32),
            pltpu.VMEM((sub,), jnp.int32),
            pltpu.VMEM((ch, 16), jnp.float32),
            pltpu.VMEM((ch, 16), jnp.float32),
            pltpu.VMEM((ch, fdim), jnp.float32),
            pltpu.VMEM((sub, fdim), jnp.float32),
            pltpu.VMEM((sub, fdim), jnp.float32),
            pltpu.VMEM_SHARED((N_NODES, fdim), jnp.float32),
        ] + [pltpu.SemaphoreType.DMA] * 8,
    )


_sc_msg1 = _make_sc_msg(HEADS * HID, HEADS, 200, 40)
_sc_msg2 = _make_sc_msg(OUT_DIM, 1, 400, 80)


def _fin_body(p_ref, out_ref):
    o = p_ref[0] + p_ref[1]
    m = jnp.max(o, axis=1, keepdims=True)
    l = o - m
    out_ref[...] = l - jnp.log(jnp.sum(jnp.exp(l), axis=1, keepdims=True))


def _fin(parts):
    n, d = parts.shape[1], parts.shape[2]
    blk = 2000
    return pl.pallas_call(
        _fin_body,
        grid=(n // blk,),
        in_specs=[pl.BlockSpec((2, blk, d), lambda i: (0, i, 0))],
        out_specs=pl.BlockSpec((blk, d), lambda i: (i, 0)),
        out_shape=jax.ShapeDtypeStruct((n, d), jnp.float32),
    )(parts)


def kernel(x, edge_index, W1, att_src1, att_dst1, W2, att_src2, att_dst2):
    src = edge_index[0].astype(jnp.int32)
    dst = edge_index[1].astype(jnp.int32)

    A1 = jnp.concatenate([_block_diag_att(att_src1), _block_diag_att(att_dst1)], axis=1)
    h1, sd1, mx1 = _tc1(x, W1, A1)
    b1 = jax.nn.leaky_relu(mx1[0, :HEADS] + mx1[0, HEADS:], NEG_SLOPE)
    b_rows1 = jnp.broadcast_to(b1[:, None], (HEADS, 16))
    zn = jnp.zeros((N_NODES,), jnp.float32)
    den1 = _sc_denom1(src, dst, sd1.T, b_rows1, zn)
    bp1 = jnp.concatenate([jnp.zeros((8,), jnp.float32), b1])[None, :]
    P1 = _p1(den1, sd1, bp1)
    Q1 = jnp.concatenate([sd1[:, HEADS:], jnp.zeros((N_NODES, 8), jnp.float32)], axis=1)
    z1 = jnp.zeros((STRIPE, HEADS * HID), jnp.float32)
    out1 = _sc_msg1(src, dst, P1, Q1, h1, z1)

    A2 = jnp.concatenate([_block_diag_att(att_src2, pad_to=8),
                          _block_diag_att(att_dst2, pad_to=8)], axis=1)
    h2, sd2, mx2 = _tc2(out1, W2, A2)
    b2 = jax.nn.leaky_relu(mx2[0, 0] + mx2[0, 8], NEG_SLOPE)
    b_rows2 = jnp.broadcast_to(b2[None, None], (1, 16))
    den2 = _sc_denom2(src, dst, sd2.T, b_rows2, zn)
    bp2 = jnp.zeros((1, 16), jnp.float32).at[0, 1].set(b2)
    P2 = _p2(den2, sd2, bp2)
    Q2 = jnp.concatenate([sd2[:, 8:9], jnp.zeros((N_NODES, 15), jnp.float32)], axis=1)
    z2 = jnp.zeros((STRIPE, OUT_DIM), jnp.float32)
    out2 = _sc_msg2(src, dst, P2, Q2, h2, z2)

    return _fin(out2)


# speedup vs baseline: 94.4024x; 1.1711x over previous
import functools

import jax
import jax.numpy as jnp
from jax import lax
from jax.experimental import pallas as pl
from jax.experimental.pallas import tpu as pltpu, tpu_sc as plsc

N_NODES = 10000
N_EDGES = 320000
IN_DIM = 128
HID = 16
HEADS = 8
OUT_DIM = 64
NEG_SLOPE = 0.2

NC = 2
NS = 16
NW = NC * NS
EPW = N_EDGES // NW
CH = 200
NCHUNK = EPW // CH
STRIPE = 632

_MESH = plsc.VectorSubcoreMesh(core_axis_name="c", subcore_axis_name="s",
                               num_cores=NC, num_subcores=NS)
_SC_PARAMS = pltpu.CompilerParams(needs_layout_passes=False,
                                  use_tc_tiling_on_sc=False)


def _block_diag_att(att, pad_to=None):
    _, H, C = att.shape
    a = att.reshape(H, C)
    eye = jnp.eye(H, dtype=att.dtype)
    out = (a[:, :, None] * eye[:, None, :]).reshape(H * C, H)
    if pad_to is not None and pad_to > H:
        out = jnp.concatenate([out, jnp.zeros((H * C, pad_to - H), att.dtype)], axis=1)
    return out


def _tc1_body(x_ref, w_ref, a_ref, h_ref, sd_ref, mx_ref):
    h = jnp.dot(x_ref[...], w_ref[...], preferred_element_type=jnp.float32)
    h_ref[...] = h
    sd = jnp.dot(h, a_ref[...], preferred_element_type=jnp.float32)
    sd_ref[...] = sd
    bmax = jnp.max(sd, axis=0, keepdims=True)

    @pl.when(pl.program_id(0) == 0)
    def _():
        mx_ref[...] = bmax

    @pl.when(pl.program_id(0) != 0)
    def _():
        mx_ref[...] = jnp.maximum(mx_ref[...], bmax)


def _tc1(x, W, A):
    n, k = x.shape
    m = W.shape[1]
    blk = 400
    return pl.pallas_call(
        _tc1_body,
        grid=(n // blk,),
        in_specs=[
            pl.BlockSpec((blk, k), lambda i: (i, 0)),
            pl.BlockSpec((k, m), lambda i: (0, 0)),
            pl.BlockSpec((m, 16), lambda i: (0, 0)),
        ],
        out_specs=[
            pl.BlockSpec((blk, m), lambda i: (i, 0)),
            pl.BlockSpec((blk, 16), lambda i: (i, 0)),
            pl.BlockSpec((1, 16), lambda i: (0, 0)),
        ],
        out_shape=[
            jax.ShapeDtypeStruct((n, m), jnp.float32),
            jax.ShapeDtypeStruct((n, 16), jnp.float32),
            jax.ShapeDtypeStruct((1, 16), jnp.float32),
        ],
    )(x, W, A)


def _tc2_body(p_ref, w_ref, a_ref, h_ref, sd_ref, mx_ref):
    hin = p_ref[0] + p_ref[1]
    hin = jnp.where(hin > 0, hin, jnp.exp(jnp.minimum(hin, 0.0)) - 1.0)
    h = jnp.dot(hin, w_ref[...], preferred_element_type=jnp.float32)
    h_ref[...] = h
    sd = jnp.dot(h, a_ref[...], preferred_element_type=jnp.float32)
    sd_ref[...] = sd
    bmax = jnp.max(sd, axis=0, keepdims=True)

    @pl.when(pl.program_id(0) == 0)
    def _():
        mx_ref[...] = bmax

    @pl.when(pl.program_id(0) != 0)
    def _():
        mx_ref[...] = jnp.maximum(mx_ref[...], bmax)


def _tc2(parts, W, A):
    n = parts.shape[1]
    k = parts.shape[2]
    m = W.shape[1]
    blk = 400
    return pl.pallas_call(
        _tc2_body,
        grid=(n // blk,),
        in_specs=[
            pl.BlockSpec((2, blk, k), lambda i: (0, i, 0)),
            pl.BlockSpec((k, m), lambda i: (0, 0)),
            pl.BlockSpec((m, 16), lambda i: (0, 0)),
        ],
        out_specs=[
            pl.BlockSpec((blk, m), lambda i: (i, 0)),
            pl.BlockSpec((blk, 16), lambda i: (i, 0)),
            pl.BlockSpec((1, 16), lambda i: (0, 0)),
        ],
        out_shape=[
            jax.ShapeDtypeStruct((n, m), jnp.float32),
            jax.ShapeDtypeStruct((n, 16), jnp.float32),
            jax.ShapeDtypeStruct((1, 16), jnp.float32),
        ],
    )(parts, W, A)


def _p1_body(den_ref, sd_ref, bp_ref, p_ref):
    d8 = jnp.sum(den_ref[...], axis=0)
    r = -jnp.log(d8 + 1e-16)
    p_ref[...] = jnp.concatenate([sd_ref[:, :8], r.T], axis=1) - bp_ref[...]


def _p1(den_parts, sd, bp):
    n = sd.shape[0]
    return pl.pallas_call(
        _p1_body,
        out_shape=jax.ShapeDtypeStruct((n, 16), jnp.float32),
    )(den_parts, sd, bp)


def _p2_body(den_ref, sd_ref, bp_ref, p_ref):
    d = jnp.sum(den_ref[...], axis=0)
    r = -jnp.log(d + 1e-16)
    bn = d.shape[0]
    p = jnp.concatenate(
        [sd_ref[:, 0:1], r[:, None], jnp.zeros((bn, 14), jnp.float32)], axis=1)
    p_ref[...] = p - bp_ref[...]


def _p2(den_parts, sd, bp):
    n = sd.shape[0]
    return pl.pallas_call(
        _p2_body,
        out_shape=jax.ShapeDtypeStruct((n, 16), jnp.float32),
    )(den_parts, sd, bp)


@functools.partial(
    pl.kernel,
    out_type=jax.ShapeDtypeStruct((NW, HEADS, N_NODES), jnp.float32),
    mesh=_MESH,
    compiler_params=_SC_PARAMS,
    scratch_types=[
        pltpu.VMEM((EPW,), jnp.int32),
        pltpu.VMEM((EPW,), jnp.int32),
        pltpu.VMEM((2, N_NODES), jnp.float32),
        pltpu.VMEM((2, N_NODES), jnp.float32),
        pltpu.VMEM((N_NODES,), jnp.float32),
        pltpu.VMEM((HEADS, 16), jnp.float32),
        pltpu.SemaphoreType.DMA,
        pltpu.SemaphoreType.DMA,
    ],
)
def _sc_denom1(src_hbm, dst_hbm, sdT_hbm, b_hbm, zn_hbm, den_out,
               src_v, dst_v, s_tab, d_tab, den_tab, b_v, tsem0, tsem1):
    cid = lax.axis_index("c")
    sid = lax.axis_index("s")
    wid = sid * NC + cid
    base = wid * EPW
    tsems = [tsem0, tsem1]

    def tab_descs(h, b):
        return (pltpu.make_async_copy(sdT_hbm.at[h], s_tab.at[b], tsems[b]),
                pltpu.make_async_copy(sdT_hbm.at[h + HEADS], d_tab.at[b], tsems[b]))

    for d in tab_descs(0, 0):
        d.start()
    pltpu.sync_copy(b_hbm, b_v)
    pltpu.sync_copy(src_hbm.at[pl.ds(base, EPW)], src_v)
    pltpu.sync_copy(dst_hbm.at[pl.ds(base, EPW)], dst_v)

    for h in range(HEADS):
        b = h % 2
        if h + 1 < HEADS:
            for d in tab_descs(h + 1, 1 - b):
                d.start()
        pltpu.sync_copy(zn_hbm, den_tab)
        for d in tab_descs(h, b):
            d.wait()
        bh = b_v[h, :]

        @plsc.parallel_loop(0, EPW // 16, unroll=4)
        def _g(g, _b=b, _bh=bh):
            di = dst_v[pl.ds(g * 16, 16)]
            sj = src_v[pl.ds(g * 16, 16)]
            t = plsc.load_gather(s_tab.at[_b], [di]) + plsc.load_gather(d_tab.at[_b], [sj])
            al = jnp.where(t >= 0, t, t * NEG_SLOPE)
            e = jnp.exp(al - _bh)
            plsc.addupdate_scatter(den_tab, [di], e)

        pltpu.sync_copy(den_tab, den_out.at[wid, h])


@functools.partial(
    pl.kernel,
    out_type=jax.ShapeDtypeStruct((NW, N_NODES), jnp.float32),
    mesh=_MESH,
    compiler_params=_SC_PARAMS,
    scratch_types=[
        pltpu.VMEM((EPW,), jnp.int32),
        pltpu.VMEM((EPW,), jnp.int32),
        pltpu.VMEM((N_NODES,), jnp.float32),
        pltpu.VMEM((N_NODES,), jnp.float32),
        pltpu.VMEM((N_NODES,), jnp.float32),
        pltpu.VMEM((16,), jnp.float32),
    ],
)
def _sc_denom2(src_hbm, dst_hbm, sdT_hbm, b_hbm, zn_hbm, den_out,
               src_v, dst_v, s_tab, d_tab, den_tab, b_v):
    cid = lax.axis_index("c")
    sid = lax.axis_index("s")
    wid = sid * NC + cid
    base = wid * EPW
    pltpu.sync_copy(src_hbm.at[pl.ds(base, EPW)], src_v)
    pltpu.sync_copy(dst_hbm.at[pl.ds(base, EPW)], dst_v)
    pltpu.sync_copy(sdT_hbm.at[0], s_tab)
    pltpu.sync_copy(sdT_hbm.at[8], d_tab)
    pltpu.sync_copy(b_hbm.at[0], b_v)
    pltpu.sync_copy(zn_hbm, den_tab)
    bh = b_v[...]

    @plsc.parallel_loop(0, EPW // 16, unroll=4)
    def _g(g):
        di = dst_v[pl.ds(g * 16, 16)]
        sj = src_v[pl.ds(g * 16, 16)]
        t = plsc.load_gather(s_tab, [di]) + plsc.load_gather(d_tab, [sj])
        al = jnp.where(t >= 0, t, t * NEG_SLOPE)
        e = jnp.exp(al - bh)
        plsc.addupdate_scatter(den_tab, [di], e)

    pltpu.sync_copy(den_tab, den_out.at[wid])


def _make_sc_msg(fdim, nheads, ch, sub):
    nsub = ch // sub
    nchunk = EPW // ch
    assert nsub == 5 and ch % sub == 0 and EPW % ch == 0
    offs = sorted(set(list(range(0, sub - 15, 16)) + [sub - 16]))

    def body(src_hbm, dst_hbm, p_hbm, q_hbm, h_hbm, z_hbm, out_hbm,
             sch, dch, didx0, didx1, didx2, didx3, didx4, prow, qrow, hrows,
             msg0, msg1, acc, esem, g0, g1, g2, g3, g4, ssem0, ssem1):
        cid = lax.axis_index("c")
        sid = lax.axis_index("s")
        wid = sid * NC + cid
        base = wid * EPW
        s0 = jnp.minimum(sid * STRIPE, N_NODES - STRIPE)
        pltpu.sync_copy(z_hbm, acc.at[pl.ds(s0, STRIPE)])
        plsc.subcore_barrier()

        didxs = [didx0, didx1, didx2, didx3, didx4]
        gsems = [g0, g1, g2, g3, g4]
        lane = lax.iota(jnp.int32, 16)
        sh_idx = (lane & 7) + 8

        def edge_descs(c):
            return (pltpu.make_async_copy(src_hbm.at[pl.ds(base + c * ch, ch)], sch, esem),
                    pltpu.make_async_copy(dst_hbm.at[pl.ds(base + c * ch, ch)], dch, esem))

        def gather_descs(r):
            rs = pl.ds(r * sub, sub)
            return (pltpu.make_async_copy(p_hbm.at[didxs[r]], prow.at[rs], gsems[r]),
                    pltpu.make_async_copy(q_hbm.at[sch.at[rs]], qrow.at[rs], gsems[r]),
                    pltpu.make_async_copy(h_hbm.at[sch.at[rs]], hrows.at[rs], gsems[r]))

        for d in edge_descs(0):
            d.start()

        @pl.loop(0, nchunk)
        def _chunk(c):
            for d in edge_descs(c):
                d.wait()
            for r in range(nsub):
                for k in offs:
                    didxs[r][pl.ds(k, 16)] = dch[pl.ds(r * sub + k, 16)]
            for r in range(nsub):
                for d in gather_descs(r):
                    d.start()

            @pl.when(c + 1 < nchunk)
            def _():
                for d in edge_descs(c + 1):
                    d.start()

            sdescs = []
            for r in range(nsub):
                mb, ssem = (msg0, ssem0) if r % 2 == 0 else (msg1, ssem1)
                for d in gather_descs(r):
                    d.wait()
                if r >= 2:
                    sdescs[r - 2].wait()

                @plsc.parallel_loop(0, sub, unroll=4)
                def _e(e, _r=r, _mb=mb):
                    ge = _r * sub + e
                    t = prow[ge, :] + qrow[ge, :]
                    u = jnp.where(t >= 0, t, t * NEG_SLOPE)
                    if nheads == 8:
                        lu = t.at[sh_idx].get(mode="promise_in_bounds")
                        a16 = jnp.exp(u + lu)
                        for h in range(8):
                            ah = a16.at[lane * 0 + h].get(mode="promise_in_bounds")
                            _mb[e, pl.ds(h * 16, 16)] = ah * hrows[ge, pl.ds(h * 16, 16)]
                    else:
                        a0 = u.at[lane * 0].get(mode="promise_in_bounds")
                        l0 = t.at[lane * 0 + 1].get(mode="promise_in_bounds")
                        a16 = jnp.exp(a0 + l0)
                        for qd in range(fdim // 16):
                            _mb[e, pl.ds(qd * 16, 16)] = a16 * hrows[ge, pl.ds(qd * 16, 16)]

                sd = pltpu.make_async_copy(mb, acc.at[didxs[r]], ssem)
                sd.start(add=True)
                sdescs.append(sd)
            sdescs[nsub - 2].wait()
            sdescs[nsub - 1].wait()

        plsc.subcore_barrier()
        pltpu.sync_copy(acc.at[pl.ds(s0, STRIPE)], out_hbm.at[cid, pl.ds(s0, STRIPE)])

    return pl.kernel(
        body,
        out_type=jax.ShapeDtypeStruct((NC, N_NODES, fdim), jnp.float32),
        mesh=_MESH,
        compiler_params=_SC_PARAMS,
        scratch_types=[
            pltpu.VMEM((ch,), jnp.int32),
            pltpu.VMEM((ch,), jnp.int32),
            pltpu.VMEM((sub,), jnp.int32),
            pltpu.VMEM((sub,), jnp.int32),
            pltpu.VMEM((sub,), jnp.int32),
            pltpu.VMEM((sub,), jnp.int32),
            pltpu.VMEM((sub,), jnp.int32),
            pltpu.VMEM((ch, 16), jnp.float32),
            pltpu.VMEM((ch, 16), jnp.float32),
            pltpu.VMEM((ch, fdim), jnp.float32),
            pltpu.VMEM((sub, fdim), jnp.float32),
            pltpu.VMEM((sub, fdim), jnp.float32),
            pltpu.VMEM_SHARED((N_NODES, fdim), jnp.float32),
        ] + [pltpu.SemaphoreType.DMA] * 8,
    )


_sc_msg1 = _make_sc_msg(HEADS * HID, HEADS, 200, 40)
_sc_msg2 = _make_sc_msg(OUT_DIM, 1, 400, 80)


def _fin_body(p_ref, out_ref):
    o = p_ref[0] + p_ref[1]
    m = jnp.max(o, axis=1, keepdims=True)
    l = o - m
    out_ref[...] = l - jnp.log(jnp.sum(jnp.exp(l), axis=1, keepdims=True))


def _fin(parts):
    n, d = parts.shape[1], parts.shape[2]
    blk = 2000
    return pl.pallas_call(
        _fin_body,
        grid=(n // blk,),
        in_specs=[pl.BlockSpec((2, blk, d), lambda i: (0, i, 0))],
        out_specs=pl.BlockSpec((blk, d), lambda i: (i, 0)),
        out_shape=jax.ShapeDtypeStruct((n, d), jnp.float32),
    )(parts)


def kernel(x, edge_index, W1, att_src1, att_dst1, W2, att_src2, att_dst2):
    src = edge_index[0].astype(jnp.int32)
    dst = edge_index[1].astype(jnp.int32)

    A1 = jnp.concatenate([_block_diag_att(att_src1), _block_diag_att(att_dst1)], axis=1)
    h1, sd1, mx1 = _tc1(x, W1, A1)
    b1 = jax.nn.leaky_relu(mx1[0, :HEADS] + mx1[0, HEADS:], NEG_SLOPE)
    b_rows1 = jnp.broadcast_to(b1[:, None], (HEADS, 16))
    zn = jnp.zeros((N_NODES,), jnp.float32)
    den1 = _sc_denom1(src, dst, sd1.T, b_rows1, zn)
    bp1 = jnp.concatenate([jnp.zeros((8,), jnp.float32), b1])[None, :]
    P1 = _p1(den1, sd1, bp1)
    Q1 = jnp.concatenate([sd1[:, HEADS:], jnp.zeros((N_NODES, 8), jnp.float32)], axis=1)
    z1 = jnp.zeros((STRIPE, HEADS * HID), jnp.float32)
    out1 = _sc_msg1(src, dst, P1, Q1, h1, z1)

    A2 = jnp.concatenate([_block_diag_att(att_src2, pad_to=8),
                          _block_diag_att(att_dst2, pad_to=8)], axis=1)
    h2, sd2, mx2 = _tc2(out1, W2, A2)
    b2 = jax.nn.leaky_relu(mx2[0, 0] + mx2[0, 8], NEG_SLOPE)
    b_rows2 = jnp.broadcast_to(b2[None, None], (1, 16))
    den2 = _sc_denom2(src, dst, sd2.T, b_rows2, zn)
    bp2 = jnp.zeros((1, 16), jnp.float32).at[0, 1].set(b2)
    P2 = _p2(den2, sd2, bp2)
    Q2 = jnp.concatenate([sd2[:, 8:9], jnp.zeros((N_NODES, 15), jnp.float32)], axis=1)
    z2 = jnp.zeros((STRIPE, OUT_DIM), jnp.float32)
    out2 = _sc_msg2(src, dst, P2, Q2, h2, z2)

    return _fin(out2)
